# Initial kernel scaffold; baseline (speedup 1.0000x reference)
import jax
import jax.numpy as jnp
from jax.experimental import pallas as pl


def kernel(h, edge_index, coord, vel, edge_attr, We1, be1, We2, be2, Wn1, bn1, Wn2, bn2, Wc1, bc1, Wc2, Wv1, bv1, Wv2, bv2):
    raise NotImplementedError("write your pallas kernel here")


# speedup vs baseline: 2.6220x; 2.6220x over previous
import functools

import jax
import jax.numpy as jnp
from jax import lax
from jax.experimental import pallas as pl
from jax.experimental.pallas import tpu as pltpu
from jax.experimental.pallas import tpu_sc as plsc

F32 = jnp.float32


def _pre_body(h_ref, c16_ref, w_ref, b_ref, out_ref):
    proj = jnp.dot(h_ref[...], w_ref[0], preferred_element_type=F32) + b_ref[0]
    out_ref[0, :, 0:128] = proj
    out_ref[0, :, 128:144] = c16_ref[...]


def _precompute_tables(h, c16, w_stk, b_stk, N, BN):
    nb = N // BN
    return pl.pallas_call(
        _pre_body,
        grid=(2, nb),
        in_specs=[
            pl.BlockSpec((BN, 128), lambda j, i: (i, 0)),
            pl.BlockSpec((BN, 16), lambda j, i: (i, 0)),
            pl.BlockSpec((1, 128, 128), lambda j, i: (j, 0, 0)),
            pl.BlockSpec((1, 1, 128), lambda j, i: (j, 0, 0)),
        ],
        out_specs=pl.BlockSpec((1, BN, 144), lambda j, i: (j, i, 0)),
        out_shape=jax.ShapeDtypeStruct((2, N, 144), F32),
    )(h, c16, w_stk, b_stk)


def _gather_edges(table2, cat_idx, E2, C):
    NW = 32
    per_w = E2 // NW
    iters = per_w // C
    mesh = plsc.VectorSubcoreMesh(core_axis_name="c", subcore_axis_name="s")

    @functools.partial(
        pl.kernel,
        mesh=mesh,
        out_type=jax.ShapeDtypeStruct((E2, 144), F32),
        scratch_types=[
            pltpu.VMEM((C,), jnp.int32),
            pltpu.VMEM((C, 144), F32),
            pltpu.SemaphoreType.DMA,
        ],
        compiler_params=pltpu.CompilerParams(use_tc_tiling_on_sc=False),
    )
    def gather_k(table_hbm, idx_hbm, out_hbm, idx_v, rows_v, sem):
        wid = lax.axis_index("s") * 2 + lax.axis_index("c")
        base = wid * per_w

        def body(i, carry):
            off = base + i * C
            pltpu.sync_copy(idx_hbm.at[pl.ds(off, C)], idx_v)
            pltpu.async_copy(table_hbm.at[idx_v], rows_v, sem).wait()
            pltpu.sync_copy(rows_v, out_hbm.at[pl.ds(off, C)])
            return carry

        lax.fori_loop(0, iters, body, 0)

    return gather_k(table2, cat_idx)


def _edge_body(gr_ref, gc_ref, ea_ref, we1r_ref, we1et_ref, we2t_ref, be2_ref,
               wc1t_ref, bc1_ref, wc2t_ref, feat_ref, tr_ref):
    gr = gr_ref[...]
    gc = gc_ref[...]
    a = gr[:, 0:128] + gc[:, 0:128]
    cd = gr[:, 128:144] - gc[:, 128:144]
    radial = jnp.sum(cd * cd, axis=1, keepdims=True)
    ef = a + radial * we1r_ref[...] + jnp.dot(
        ea_ref[...], we1et_ref[...], preferred_element_type=F32)
    ef = jnp.maximum(ef, 0.0)
    feat = jnp.maximum(
        jnp.dot(ef, we2t_ref[...], preferred_element_type=F32) + be2_ref[...],
        0.0)
    feat_ref[...] = feat
    cm = jnp.maximum(
        jnp.dot(feat, wc1t_ref[...], preferred_element_type=F32) + bc1_ref[...],
        0.0)
    m = jnp.dot(cm, wc2t_ref[...], preferred_element_type=F32)
    tr = jnp.clip(cd * m, -100.0, 100.0)
    one3 = (lax.broadcasted_iota(jnp.int32, (1, 16), 1) == 3).astype(F32)
    tr_ref[...] = tr + one3


def _edge_mlp(gflat, ea, we1r, we1et, we2t, be2, wc1t, bc1, wc2t, E, BE):
    nb = E // BE
    wspec = lambda shape: pl.BlockSpec(shape, lambda i: tuple(0 for _ in shape))
    return pl.pallas_call(
        _edge_body,
        grid=(nb,),
        in_specs=[
            pl.BlockSpec((BE, 144), lambda i: (i, 0)),
            pl.BlockSpec((BE, 144), lambda i: (i + nb, 0)),
            pl.BlockSpec((BE, 16), lambda i: (i, 0)),
            wspec((1, 128)),
            wspec((16, 128)),
            wspec((128, 128)),
            wspec((1, 128)),
            wspec((128, 128)),
            wspec((1, 128)),
            wspec((128, 1)),
        ],
        out_specs=[
            pl.BlockSpec((BE, 128), lambda i: (i, 0)),
            pl.BlockSpec((BE, 16), lambda i: (i, 0)),
        ],
        out_shape=[
            jax.ShapeDtypeStruct((E, 128), F32),
            jax.ShapeDtypeStruct((E, 16), F32),
        ],
    )(gflat, gflat, ea, we1r, we1et, we2t, be2, wc1t, bc1, wc2t)


def _segment_sums(feat, tr16, cat_rc, z128, z16, N, E, C):
    NT = 16
    per_t = E // NT
    iters = per_t // C
    rows_t = N // NT
    mesh = plsc.VectorSubcoreMesh(core_axis_name="c", subcore_axis_name="s")

    @functools.partial(
        pl.kernel,
        mesh=mesh,
        out_type=(
            jax.ShapeDtypeStruct((N, 128), F32),
            jax.ShapeDtypeStruct((N, 128), F32),
            jax.ShapeDtypeStruct((N, 16), F32),
        ),
        scratch_types=[
            pltpu.VMEM((C,), jnp.int32),
            pltpu.VMEM((C, 128), F32),
            pltpu.VMEM((C, 16), F32),
            pltpu.VMEM_SHARED((N, 128), F32),
            pltpu.VMEM_SHARED((N, 16), F32),
        ],
        compiler_params=pltpu.CompilerParams(use_tc_tiling_on_sc=False),
    )
    def scatter_k(feat_hbm, tr_hbm, idx_hbm, z128_hbm, z16_hbm,
                  outr_hbm, outc_hbm, outt_hbm,
                  idx_v, feat_v, tr_v, accf, acct):
        c = lax.axis_index("c")
        s = lax.axis_index("s")
        nslice = pl.ds(s * rows_t, rows_t)
        pltpu.sync_copy(z128_hbm.at[nslice], accf.at[nslice])

        @pl.when(c == 0)
        def _():
            pltpu.sync_copy(z16_hbm.at[nslice], acct.at[nslice])

        plsc.subcore_barrier()

        def body(i, carry):
            ebase = s * per_t + i * C
            pltpu.sync_copy(idx_hbm.at[pl.ds(c * E + ebase, C)], idx_v)
            pltpu.sync_copy(feat_hbm.at[pl.ds(ebase, C)], feat_v)
            pltpu.sync_copy(feat_v, accf.at[idx_v], add=True)

            @pl.when(c == 0)
            def _():
                pltpu.sync_copy(tr_hbm.at[pl.ds(ebase, C)], tr_v)
                pltpu.sync_copy(tr_v, acct.at[idx_v], add=True)

            return carry

        lax.fori_loop(0, iters, body, 0)
        plsc.subcore_barrier()

        @pl.when(c == 0)
        def _():
            pltpu.sync_copy(accf.at[nslice], outr_hbm.at[nslice])
            pltpu.sync_copy(acct.at[nslice], outt_hbm.at[nslice])

        @pl.when(c == 1)
        def _():
            pltpu.sync_copy(accf.at[nslice], outc_hbm.at[nslice])

    return scatter_k(feat, tr16, cat_rc, z128, z16)


def _node_body(aggc_ref, h_ref, aggr_ref, aggt_ref, coord_ref, vel_ref,
               wn1ot_ref, wn1ht_ref, wn1at_ref, bn1_ref, wn2t_ref, bn2_ref,
               wv1t_ref, bv1_ref, wv2t_ref, bv2_ref,
               hout_ref, cout_ref, vout_ref):
    h = h_ref[...]
    aggt = aggt_ref[...]
    cnt = jnp.maximum(aggt[:, 3:4], 1.0)
    f = aggt[:, 0:3] / cnt
    hv = jnp.maximum(
        jnp.dot(h, wv1t_ref[...], preferred_element_type=F32) + bv1_ref[...],
        0.0)
    phi = jnp.dot(hv, wv2t_ref[...], preferred_element_type=F32) + bv2_ref[...]
    vel_new = phi * vel_ref[...] + f
    vout_ref[...] = vel_new
    cout_ref[...] = coord_ref[...] + vel_new
    pre = (jnp.dot(aggc_ref[...], wn1ot_ref[...], preferred_element_type=F32)
           + jnp.dot(h, wn1ht_ref[...], preferred_element_type=F32)
           + jnp.dot(aggr_ref[...], wn1at_ref[...], preferred_element_type=F32)
           + bn1_ref[...])
    hn = jnp.dot(jnp.maximum(pre, 0.0), wn2t_ref[...],
                 preferred_element_type=F32) + bn2_ref[...]
    hout_ref[...] = h + hn


def _node_update(aggc, h, aggr, aggt, coord, vel, wn1ot, wn1ht, wn1at, bn1,
                 wn2t, bn2, wv1t, bv1, wv2t, bv2, N, BN):
    nb = N // BN
    wspec = lambda shape: pl.BlockSpec(shape, lambda i: tuple(0 for _ in shape))
    return pl.pallas_call(
        _node_body,
        grid=(nb,),
        in_specs=[
            pl.BlockSpec((BN, 128), lambda i: (i, 0)),
            pl.BlockSpec((BN, 128), lambda i: (i, 0)),
            pl.BlockSpec((BN, 128), lambda i: (i, 0)),
            pl.BlockSpec((BN, 16), lambda i: (i, 0)),
            pl.BlockSpec((BN, 3), lambda i: (i, 0)),
            pl.BlockSpec((BN, 3), lambda i: (i, 0)),
            wspec((128, 128)),
            wspec((128, 128)),
            wspec((128, 128)),
            wspec((1, 128)),
            wspec((128, 128)),
            wspec((1, 128)),
            wspec((128, 128)),
            wspec((1, 128)),
            wspec((128, 1)),
            wspec((1, 1)),
        ],
        out_specs=[
            pl.BlockSpec((BN, 128), lambda i: (i, 0)),
            pl.BlockSpec((BN, 3), lambda i: (i, 0)),
            pl.BlockSpec((BN, 3), lambda i: (i, 0)),
        ],
        out_shape=[
            jax.ShapeDtypeStruct((N, 128), F32),
            jax.ShapeDtypeStruct((N, 3), F32),
            jax.ShapeDtypeStruct((N, 3), F32),
        ],
    )(aggc, h, aggr, aggt, coord, vel, wn1ot, wn1ht, wn1at, bn1, wn2t, bn2,
      wv1t, bv1, wv2t, bv2)


def kernel(h, edge_index, coord, vel, edge_attr, We1, be1, We2, be2,
           Wn1, bn1, Wn2, bn2, Wc1, bc1, Wc2, Wv1, bv1, Wv2, bv2):
    N, D = h.shape
    E = edge_index.shape[1]
    BN = 2000
    BE = 2000
    C = 80

    row = edge_index[0].astype(jnp.int32)
    col = edge_index[1].astype(jnp.int32)

    w_stk = jnp.stack([We1[:, :D].T, We1[:, D:2 * D].T])
    b_stk = jnp.stack([jnp.zeros((1, 128), F32), be1.reshape(1, 128)])
    we1r = We1[:, 2 * D:2 * D + 1].T
    we1et = We1[:, 2 * D + 1:].T
    c16 = jnp.pad(coord, ((0, 0), (0, 13)))

    tables = _precompute_tables(h, c16, w_stk, b_stk, N, BN)
    table2 = tables.reshape(2 * N, 144)

    cat_idx = jnp.concatenate([row, col + N])
    gflat = _gather_edges(table2, cat_idx, 2 * E, C)

    feat, tr16 = _edge_mlp(gflat, edge_attr, we1r, we1et, We2.T,
                           be2.reshape(1, 128), Wc1.T, bc1.reshape(1, 128),
                           Wc2.T, E, BE)

    cat_rc = jnp.concatenate([row, col])
    z128 = jnp.zeros((N, 128), F32)
    z16 = jnp.zeros((N, 16), F32)
    aggr, aggc, aggt = _segment_sums(feat, tr16, cat_rc, z128, z16, N, E, C)

    h_new, coord_new, vel_new = _node_update(
        aggc, h, aggr, aggt, coord, vel,
        Wn1[:, :128].T, Wn1[:, 128:256].T, Wn1[:, 256:].T,
        bn1.reshape(1, 128), Wn2.T, bn2.reshape(1, 128),
        Wv1.T, bv1.reshape(1, 128), Wv2.T, bv2.reshape(1, 1), N, BN)
    return h_new, coord_new, vel_new


# speedup vs baseline: 3.0628x; 1.1681x over previous
import functools

import jax
import jax.numpy as jnp
from jax import lax
from jax.experimental import pallas as pl
from jax.experimental.pallas import tpu as pltpu
from jax.experimental.pallas import tpu_sc as plsc

F32 = jnp.float32


def _pre_body(h_ref, w_ref, b_ref, out_ref):
    out_ref[...] = jnp.dot(h_ref[...], w_ref[0],
                           preferred_element_type=F32) + b_ref[0]


def _precompute_tables(h, w_stk, b_stk, N, BN):
    nb = N // BN
    return pl.pallas_call(
        _pre_body,
        grid=(2, nb),
        in_specs=[
            pl.BlockSpec((BN, 128), lambda j, i: (i, 0)),
            pl.BlockSpec((1, 128, 128), lambda j, i: (j, 0, 0)),
            pl.BlockSpec((1, 1, 128), lambda j, i: (j, 0, 0)),
        ],
        out_specs=pl.BlockSpec((BN, 128), lambda j, i: (j * nb + i, 0)),
        out_shape=jax.ShapeDtypeStruct((2 * N, 128), F32),
    )(h, w_stk, b_stk)


def _gather_proj(table2, cat_idx, E2, C):
    NW = 32
    per_w = E2 // NW
    iters = per_w // C
    mesh = plsc.VectorSubcoreMesh(core_axis_name="c", subcore_axis_name="s")

    @functools.partial(
        pl.kernel,
        mesh=mesh,
        out_type=jax.ShapeDtypeStruct((E2, 128), F32),
        scratch_types=[
            pltpu.VMEM((C,), jnp.int32),
            pltpu.VMEM((C, 128), F32),
            pltpu.SemaphoreType.DMA,
        ],
    )
    def gather_k(table_hbm, idx_hbm, out_hbm, idx_v, rows_v, sem):
        wid = lax.axis_index("s") * 2 + lax.axis_index("c")
        base = wid * per_w

        def body(i, carry):
            off = base + i * C
            pltpu.sync_copy(idx_hbm.at[pl.ds(off, C)], idx_v)
            pltpu.async_copy(table_hbm.at[idx_v], rows_v, sem).wait()
            pltpu.sync_copy(rows_v, out_hbm.at[pl.ds(off, C)])
            return carry

        lax.fori_loop(0, iters, body, 0)

    return gather_k(table2, cat_idx)


def _gather_coords(c16, cat_rc, E2, C):
    NW = 32
    per_w = E2 // NW
    iters = per_w // C
    mesh = plsc.VectorSubcoreMesh(core_axis_name="c", subcore_axis_name="s")

    @functools.partial(
        pl.kernel,
        mesh=mesh,
        out_type=jax.ShapeDtypeStruct((E2, 16), F32),
        scratch_types=[
            pltpu.VMEM((C,), jnp.int32),
            pltpu.VMEM((C, 16), F32),
            pltpu.SemaphoreType.DMA,
        ],
        compiler_params=pltpu.CompilerParams(use_tc_tiling_on_sc=False),
    )
    def gatherc_k(tab_hbm, idx_hbm, out_hbm, idx_v, rows_v, sem):
        wid = lax.axis_index("s") * 2 + lax.axis_index("c")
        base = wid * per_w

        def body(i, carry):
            off = base + i * C
            pltpu.sync_copy(idx_hbm.at[pl.ds(off, C)], idx_v)
            pltpu.async_copy(tab_hbm.at[idx_v], rows_v, sem).wait()
            pltpu.sync_copy(rows_v, out_hbm.at[pl.ds(off, C)])
            return carry

        lax.fori_loop(0, iters, body, 0)

    return gatherc_k(c16, cat_rc)


def _edge_body(gpr_ref, gpc_ref, gcr_ref, gcc_ref, ea_ref, we1r_ref,
               we1et_ref, we2t_ref, be2_ref, wc1t_ref, bc1_ref, wc2t_ref,
               feat_ref, tr_ref):
    a = gpr_ref[...] + gpc_ref[...]
    cd = gcr_ref[...] - gcc_ref[...]
    radial = jnp.sum(cd * cd, axis=1, keepdims=True)
    ef = a + radial * we1r_ref[...] + jnp.dot(
        ea_ref[...], we1et_ref[...], preferred_element_type=F32)
    ef = jnp.maximum(ef, 0.0)
    feat = jnp.maximum(
        jnp.dot(ef, we2t_ref[...], preferred_element_type=F32) + be2_ref[...],
        0.0)
    feat_ref[...] = feat
    cm = jnp.maximum(
        jnp.dot(feat, wc1t_ref[...], preferred_element_type=F32) + bc1_ref[...],
        0.0)
    m = jnp.dot(cm, wc2t_ref[...], preferred_element_type=F32)
    tr = jnp.clip(cd * m, -100.0, 100.0)
    one3 = (lax.broadcasted_iota(jnp.int32, (1, 16), 1) == 3).astype(F32)
    tr_ref[...] = tr + one3


def _edge_mlp(gp, g16, ea, we1r, we1et, we2t, be2, wc1t, bc1, wc2t, E, BE):
    nb = E // BE
    wspec = lambda shape: pl.BlockSpec(shape, lambda i: tuple(0 for _ in shape))
    return pl.pallas_call(
        _edge_body,
        grid=(nb,),
        in_specs=[
            pl.BlockSpec((BE, 128), lambda i: (i, 0)),
            pl.BlockSpec((BE, 128), lambda i: (i + nb, 0)),
            pl.BlockSpec((BE, 16), lambda i: (i, 0)),
            pl.BlockSpec((BE, 16), lambda i: (i + nb, 0)),
            pl.BlockSpec((BE, 16), lambda i: (i, 0)),
            wspec((1, 128)),
            wspec((16, 128)),
            wspec((128, 128)),
            wspec((1, 128)),
            wspec((128, 128)),
            wspec((1, 128)),
            wspec((128, 1)),
        ],
        out_specs=[
            pl.BlockSpec((BE, 128), lambda i: (i, 0)),
            pl.BlockSpec((BE, 16), lambda i: (i, 0)),
        ],
        out_shape=[
            jax.ShapeDtypeStruct((E, 128), F32),
            jax.ShapeDtypeStruct((E, 16), F32),
        ],
    )(gp, gp, g16, g16, ea, we1r, we1et, we2t, be2, wc1t, bc1, wc2t)


def _segment_feat(feat, cat_rc, z128, N, E, C):
    NT = 16
    per_t = E // NT
    iters = per_t // C
    rows_t = N // NT
    mesh = plsc.VectorSubcoreMesh(core_axis_name="c", subcore_axis_name="s")

    @functools.partial(
        pl.kernel,
        mesh=mesh,
        out_type=(
            jax.ShapeDtypeStruct((N, 128), F32),
            jax.ShapeDtypeStruct((N, 128), F32),
        ),
        scratch_types=[
            pltpu.VMEM((C,), jnp.int32),
            pltpu.VMEM((C, 128), F32),
            pltpu.VMEM_SHARED((N, 128), F32),
        ],
    )
    def scatter_k(feat_hbm, idx_hbm, z_hbm, outr_hbm, outc_hbm,
                  idx_v, feat_v, accf):
        c = lax.axis_index("c")
        s = lax.axis_index("s")
        base = s * 640
        tail = N - 15 * 640

        @pl.when(s < 15)
        def _():
            sl = pl.ds(base, 640)
            pltpu.sync_copy(z_hbm.at[sl], accf.at[sl])

        @pl.when(s == 15)
        def _():
            sl = pl.ds(base, tail)
            pltpu.sync_copy(z_hbm.at[sl], accf.at[sl])

        plsc.subcore_barrier()

        def body(i, carry):
            ebase = s * per_t + i * C
            pltpu.sync_copy(idx_hbm.at[pl.ds(c * E + ebase, C)], idx_v)
            pltpu.sync_copy(feat_hbm.at[pl.ds(ebase, C)], feat_v)
            pltpu.sync_copy(feat_v, accf.at[idx_v], add=True)
            return carry

        lax.fori_loop(0, iters, body, 0)
        plsc.subcore_barrier()

        @pl.when((c == 0) & (s < 15))
        def _():
            sl = pl.ds(base, 640)
            pltpu.sync_copy(accf.at[sl], outr_hbm.at[sl])

        @pl.when((c == 0) & (s == 15))
        def _():
            sl = pl.ds(base, tail)
            pltpu.sync_copy(accf.at[sl], outr_hbm.at[sl])

        @pl.when((c == 1) & (s < 15))
        def _():
            sl = pl.ds(base, 640)
            pltpu.sync_copy(accf.at[sl], outc_hbm.at[sl])

        @pl.when((c == 1) & (s == 15))
        def _():
            sl = pl.ds(base, tail)
            pltpu.sync_copy(accf.at[sl], outc_hbm.at[sl])

    return scatter_k(feat, cat_rc, z128)


def _segment_tr(tr16, row_idx, z16, N, E, C):
    NT = 16
    half = E // 2
    per_t = half // NT
    iters = per_t // C
    rows_t = N // NT
    mesh = plsc.VectorSubcoreMesh(core_axis_name="c", subcore_axis_name="s")

    @functools.partial(
        pl.kernel,
        mesh=mesh,
        out_type=jax.ShapeDtypeStruct((2, N, 16), F32),
        scratch_types=[
            pltpu.VMEM((C,), jnp.int32),
            pltpu.VMEM((C, 16), F32),
            pltpu.VMEM_SHARED((N, 16), F32),
        ],
        compiler_params=pltpu.CompilerParams(use_tc_tiling_on_sc=False),
    )
    def scattr_k(tr_hbm, idx_hbm, z_hbm, out_hbm, idx_v, tr_v, acct):
        c = lax.axis_index("c")
        s = lax.axis_index("s")
        nslice = pl.ds(s * rows_t, rows_t)
        pltpu.sync_copy(z_hbm.at[nslice], acct.at[nslice])
        plsc.subcore_barrier()

        def body(i, carry):
            ebase = c * half + s * per_t + i * C
            pltpu.sync_copy(idx_hbm.at[pl.ds(ebase, C)], idx_v)
            pltpu.sync_copy(tr_hbm.at[pl.ds(ebase, C)], tr_v)
            pltpu.sync_copy(tr_v, acct.at[idx_v], add=True)
            return carry

        lax.fori_loop(0, iters, body, 0)
        plsc.subcore_barrier()
        pltpu.sync_copy(acct.at[nslice], out_hbm.at[c].at[nslice])

    return scattr_k(tr16, row_idx, z16)


def _node_body(aggc_ref, h_ref, aggr_ref, aggt_ref, coord_ref, vel_ref,
               wn1ot_ref, wn1ht_ref, wn1at_ref, bn1_ref, wn2t_ref, bn2_ref,
               wv1t_ref, bv1_ref, wv2t_ref, bv2_ref,
               hout_ref, cout_ref, vout_ref):
    h = h_ref[...]
    aggt = aggt_ref[0] + aggt_ref[1]
    cnt = jnp.maximum(aggt[:, 3:4], 1.0)
    f = aggt[:, 0:3] / cnt
    hv = jnp.maximum(
        jnp.dot(h, wv1t_ref[...], preferred_element_type=F32) + bv1_ref[...],
        0.0)
    phi = jnp.dot(hv, wv2t_ref[...], preferred_element_type=F32) + bv2_ref[...]
    vel_new = phi * vel_ref[...] + f
    vout_ref[...] = vel_new
    cout_ref[...] = coord_ref[...] + vel_new
    pre = (jnp.dot(aggc_ref[...], wn1ot_ref[...], preferred_element_type=F32)
           + jnp.dot(h, wn1ht_ref[...], preferred_element_type=F32)
           + jnp.dot(aggr_ref[...], wn1at_ref[...], preferred_element_type=F32)
           + bn1_ref[...])
    hn = jnp.dot(jnp.maximum(pre, 0.0), wn2t_ref[...],
                 preferred_element_type=F32) + bn2_ref[...]
    hout_ref[...] = h + hn


def _node_update(aggc, h, aggr, aggt2, coord, vel, wn1ot, wn1ht, wn1at, bn1,
                 wn2t, bn2, wv1t, bv1, wv2t, bv2, N, BN):
    nb = N // BN
    wspec = lambda shape: pl.BlockSpec(shape, lambda i: tuple(0 for _ in shape))
    return pl.pallas_call(
        _node_body,
        grid=(nb,),
        in_specs=[
            pl.BlockSpec((BN, 128), lambda i: (i, 0)),
            pl.BlockSpec((BN, 128), lambda i: (i, 0)),
            pl.BlockSpec((BN, 128), lambda i: (i, 0)),
            pl.BlockSpec((2, BN, 16), lambda i: (0, i, 0)),
            pl.BlockSpec((BN, 3), lambda i: (i, 0)),
            pl.BlockSpec((BN, 3), lambda i: (i, 0)),
            wspec((128, 128)),
            wspec((128, 128)),
            wspec((128, 128)),
            wspec((1, 128)),
            wspec((128, 128)),
            wspec((1, 128)),
            wspec((128, 128)),
            wspec((1, 128)),
            wspec((128, 1)),
            wspec((1, 1)),
        ],
        out_specs=[
            pl.BlockSpec((BN, 128), lambda i: (i, 0)),
            pl.BlockSpec((BN, 3), lambda i: (i, 0)),
            pl.BlockSpec((BN, 3), lambda i: (i, 0)),
        ],
        out_shape=[
            jax.ShapeDtypeStruct((N, 128), F32),
            jax.ShapeDtypeStruct((N, 3), F32),
            jax.ShapeDtypeStruct((N, 3), F32),
        ],
    )(aggc, h, aggr, aggt2, coord, vel, wn1ot, wn1ht, wn1at, bn1, wn2t, bn2,
      wv1t, bv1, wv2t, bv2)


def kernel(h, edge_index, coord, vel, edge_attr, We1, be1, We2, be2,
           Wn1, bn1, Wn2, bn2, Wc1, bc1, Wc2, Wv1, bv1, Wv2, bv2):
    N, D = h.shape
    E = edge_index.shape[1]
    BN = 2000
    BE = 2000
    C = 80

    row = edge_index[0].astype(jnp.int32)
    col = edge_index[1].astype(jnp.int32)

    w_stk = jnp.stack([We1[:, :D].T, We1[:, D:2 * D].T])
    b_stk = jnp.stack([jnp.zeros((1, 128), F32), be1.reshape(1, 128)])
    we1r = We1[:, 2 * D:2 * D + 1].T
    we1et = We1[:, 2 * D + 1:].T
    c16 = jnp.pad(coord, ((0, 0), (0, 13)))

    table2 = _precompute_tables(h, w_stk, b_stk, N, BN)

    cat_idx = jnp.concatenate([row, col + N])
    cat_rc = jnp.concatenate([row, col])
    gp = _gather_proj(table2, cat_idx, 2 * E, C)
    g16 = _gather_coords(c16, cat_rc, 2 * E, C)

    feat, tr16 = _edge_mlp(gp, g16, edge_attr, we1r, we1et, We2.T,
                           be2.reshape(1, 128), Wc1.T, bc1.reshape(1, 128),
                           Wc2.T, E, BE)

    z128 = jnp.zeros((N, 128), F32)
    z16 = jnp.zeros((N, 16), F32)
    aggr, aggc = _segment_feat(feat, cat_rc, z128, N, E, C)
    aggt2 = _segment_tr(tr16, row, z16, N, E, C)

    h_new, coord_new, vel_new = _node_update(
        aggc, h, aggr, aggt2, coord, vel,
        Wn1[:, :128].T, Wn1[:, 128:256].T, Wn1[:, 256:].T,
        bn1.reshape(1, 128), Wn2.T, bn2.reshape(1, 128),
        Wv1.T, bv1.reshape(1, 128), Wv2.T, bv2.reshape(1, 1), N, BN)
    return h_new, coord_new, vel_new


# speedup vs baseline: 3.2405x; 1.0580x over previous
import functools

import jax
import jax.numpy as jnp
from jax import lax
from jax.experimental import pallas as pl
from jax.experimental.pallas import tpu as pltpu
from jax.experimental.pallas import tpu_sc as plsc

F32 = jnp.float32
C = 128


def _pre_body(h_ref, w_ref, b_ref, out_ref):
    out_ref[...] = jnp.dot(h_ref[...], w_ref[0],
                           preferred_element_type=F32) + b_ref[0]


def _precompute_tables(h, w_stk, b_stk, N, BN):
    nb = N // BN
    return pl.pallas_call(
        _pre_body,
        grid=(2, nb),
        in_specs=[
            pl.BlockSpec((BN, 128), lambda j, i: (i, 0)),
            pl.BlockSpec((1, 128, 128), lambda j, i: (j, 0, 0)),
            pl.BlockSpec((1, 1, 128), lambda j, i: (j, 0, 0)),
        ],
        out_specs=pl.BlockSpec((BN, 128), lambda j, i: (j * nb + i, 0)),
        out_shape=jax.ShapeDtypeStruct((2 * N, 128), F32),
    )(h, w_stk, b_stk)


def _ring_gather(table, idx3d, E2, W, tc_tiling):
    NW = 32
    per_w = E2 // NW
    iters = per_w // C
    mesh = plsc.VectorSubcoreMesh(core_axis_name="c", subcore_axis_name="s")

    @functools.partial(
        pl.kernel,
        mesh=mesh,
        out_type=jax.ShapeDtypeStruct((E2, W), F32),
        scratch_types=[
            pltpu.VMEM((1, iters, C), jnp.int32),
            pltpu.VMEM((C, W), F32),
            pltpu.VMEM((C, W), F32),
            pltpu.SemaphoreType.DMA,
            pltpu.SemaphoreType.DMA,
            pltpu.SemaphoreType.DMA,
            pltpu.SemaphoreType.DMA,
        ],
        compiler_params=pltpu.CompilerParams(use_tc_tiling_on_sc=tc_tiling),
    )
    def gather_k(table_hbm, idx_hbm, out_hbm, idx_v, r0, r1, g0, g1, w0, w1):
        wid = lax.axis_index("s") * 2 + lax.axis_index("c")
        base = wid * per_w
        rows = (r0, r1)
        gs = (g0, g1)
        ws = (w0, w1)
        pltpu.sync_copy(idx_hbm.at[pl.ds(wid, 1)], idx_v)
        pltpu.async_copy(table_hbm.at[idx_v.at[0, 0]], rows[0], gs[0])

        def pair(g, carry):
            for b in (0, 1):
                i = 2 * g + b
                nb = 1 - b
                pltpu.make_async_copy(
                    table_hbm.at[idx_v.at[0, i]], rows[b], gs[b]).wait()

                @pl.when(i >= 1)
                def _():
                    pltpu.make_async_copy(
                        rows[nb], out_hbm.at[pl.ds(base + (i - 1) * C, C)],
                        ws[nb]).wait()

                @pl.when(i + 1 < iters)
                def _():
                    pltpu.async_copy(
                        table_hbm.at[idx_v.at[0, i + 1]], rows[nb], gs[nb])

                pltpu.async_copy(
                    rows[b], out_hbm.at[pl.ds(base + i * C, C)], ws[b])
            return carry

        lax.fori_loop(0, iters // 2, pair, 0)
        pltpu.make_async_copy(
            rows[1], out_hbm.at[pl.ds(base + (iters - 1) * C, C)], ws[1]).wait()

    return gather_k(table, idx3d)


def _edge_body(gpr_ref, gpc_ref, gcr_ref, gcc_ref, ea_ref, we1r_ref,
               we1et_ref, we2t_ref, be2_ref, wc1t_ref, bc1_ref, wc2t_ref,
               feat_ref, tr_ref):
    a = gpr_ref[...] + gpc_ref[...]
    cd = gcr_ref[...] - gcc_ref[...]
    radial = jnp.sum(cd * cd, axis=1, keepdims=True)
    ef = a + radial * we1r_ref[...] + jnp.dot(
        ea_ref[...], we1et_ref[...], preferred_element_type=F32)
    ef = jnp.maximum(ef, 0.0)
    feat = jnp.maximum(
        jnp.dot(ef, we2t_ref[...], preferred_element_type=F32) + be2_ref[...],
        0.0)
    feat_ref[...] = feat
    cm = jnp.maximum(
        jnp.dot(feat, wc1t_ref[...], preferred_element_type=F32) + bc1_ref[...],
        0.0)
    m = jnp.dot(cm, wc2t_ref[...], preferred_element_type=F32)
    tr = jnp.clip(cd * m, -100.0, 100.0)
    one3 = (lax.broadcasted_iota(jnp.int32, (1, 16), 1) == 3).astype(F32)
    tr_ref[...] = tr + one3


def _edge_mlp(gp, g16, ea, we1r, we1et, we2t, be2, wc1t, bc1, wc2t, EP, BE):
    nb = EP // BE
    wspec = lambda shape: pl.BlockSpec(shape, lambda i: tuple(0 for _ in shape))
    return pl.pallas_call(
        _edge_body,
        grid=(nb,),
        in_specs=[
            pl.BlockSpec((BE, 128), lambda i: (i, 0)),
            pl.BlockSpec((BE, 128), lambda i: (i + nb, 0)),
            pl.BlockSpec((BE, 16), lambda i: (i, 0)),
            pl.BlockSpec((BE, 16), lambda i: (i + nb, 0)),
            pl.BlockSpec((BE, 16), lambda i: (i, 0)),
            wspec((1, 128)),
            wspec((16, 128)),
            wspec((128, 128)),
            wspec((1, 128)),
            wspec((128, 128)),
            wspec((1, 128)),
            wspec((128, 1)),
        ],
        out_specs=[
            pl.BlockSpec((BE, 128), lambda i: (i, 0)),
            pl.BlockSpec((BE, 16), lambda i: (i, 0)),
        ],
        out_shape=[
            jax.ShapeDtypeStruct((EP, 128), F32),
            jax.ShapeDtypeStruct((EP, 16), F32),
        ],
    )(gp, gp, g16, g16, ea, we1r, we1et, we2t, be2, wc1t, bc1, wc2t)


def _segment_feat(feat, cat_rc3, z128, NA, EP):
    NT = 16
    per_t = EP // NT
    iters = per_t // C
    HI = iters // 2
    mesh = plsc.VectorSubcoreMesh(core_axis_name="c", subcore_axis_name="s")

    @functools.partial(
        pl.kernel,
        mesh=mesh,
        out_type=(
            jax.ShapeDtypeStruct((NA, 128), F32),
            jax.ShapeDtypeStruct((NA, 128), F32),
        ),
        scratch_types=[
            pltpu.VMEM((1, HI, C), jnp.int32),
            pltpu.VMEM((C, 128), F32),
            pltpu.VMEM((C, 128), F32),
            pltpu.SemaphoreType.DMA,
            pltpu.SemaphoreType.DMA,
            pltpu.SemaphoreType.DMA,
            pltpu.VMEM_SHARED((NA, 128), F32),
        ],
    )
    def scatter_k(feat_hbm, idx_hbm, z_hbm, outr_hbm, outc_hbm,
                  idx_v, f0, f1, fs0, fs1, ssem, accf):
        c = lax.axis_index("c")
        s = lax.axis_index("s")
        blk = c * 16 + s
        base = s * 640
        tail = NA - 15 * 640

        @pl.when(s < 15)
        def _():
            sl = pl.ds(base, 640)
            pltpu.sync_copy(z_hbm.at[sl], accf.at[sl])

        @pl.when(s == 15)
        def _():
            sl = pl.ds(base, tail)
            pltpu.sync_copy(z_hbm.at[sl], accf.at[sl])

        plsc.subcore_barrier()

        fb = (f0, f1)
        fs = (fs0, fs1)
        ebase = s * per_t

        for h in (0, 1):
            pltpu.sync_copy(idx_hbm.at[pl.ds(blk, 1), pl.ds(h * HI, HI)],
                            idx_v)
            hbase = ebase + h * HI * C
            pltpu.async_copy(feat_hbm.at[pl.ds(hbase, C)], fb[0], fs[0])

            def pair(g, carry):
                for b in (0, 1):
                    i = 2 * g + b
                    nb = 1 - b
                    pltpu.make_async_copy(
                        feat_hbm.at[pl.ds(hbase + i * C, C)],
                        fb[b], fs[b]).wait()

                    @pl.when(i + 1 < HI)
                    def _():
                        pltpu.async_copy(
                            feat_hbm.at[pl.ds(hbase + (i + 1) * C, C)],
                            fb[nb], fs[nb])

                    pltpu.sync_copy(fb[b], accf.at[idx_v.at[0, i]], add=True)
                return carry

            lax.fori_loop(0, HI // 2, pair, 0)

        plsc.subcore_barrier()

        @pl.when((c == 0) & (s < 15))
        def _():
            sl = pl.ds(base, 640)
            pltpu.sync_copy(accf.at[sl], outr_hbm.at[sl])

        @pl.when((c == 0) & (s == 15))
        def _():
            sl = pl.ds(base, tail)
            pltpu.sync_copy(accf.at[sl], outr_hbm.at[sl])

        @pl.when((c == 1) & (s < 15))
        def _():
            sl = pl.ds(base, 640)
            pltpu.sync_copy(accf.at[sl], outc_hbm.at[sl])

        @pl.when((c == 1) & (s == 15))
        def _():
            sl = pl.ds(base, tail)
            pltpu.sync_copy(accf.at[sl], outc_hbm.at[sl])

    return scatter_k(feat, cat_rc3, z128)


def _segment_tr(tr16, row3, z16, NA, EP):
    NT = 16
    half = EP // 2
    per_t = half // NT
    iters = per_t // C
    rows_t = NA // NT
    mesh = plsc.VectorSubcoreMesh(core_axis_name="c", subcore_axis_name="s")

    @functools.partial(
        pl.kernel,
        mesh=mesh,
        out_type=jax.ShapeDtypeStruct((2, NA, 16), F32),
        scratch_types=[
            pltpu.VMEM((1, iters, C), jnp.int32),
            pltpu.VMEM((C, 16), F32),
            pltpu.VMEM((C, 16), F32),
            pltpu.SemaphoreType.DMA,
            pltpu.SemaphoreType.DMA,
            pltpu.SemaphoreType.DMA,
            pltpu.VMEM_SHARED((NA, 16), F32),
        ],
        compiler_params=pltpu.CompilerParams(use_tc_tiling_on_sc=False),
    )
    def scattr_k(tr_hbm, idx_hbm, z_hbm, out_hbm,
                 idx_v, t0, t1, ts0, ts1, ssem, acct):
        c = lax.axis_index("c")
        s = lax.axis_index("s")
        blk = c * 16 + s
        nslice = pl.ds(s * rows_t, rows_t)
        pltpu.sync_copy(z_hbm.at[nslice], acct.at[nslice])
        plsc.subcore_barrier()

        tb = (t0, t1)
        ts = (ts0, ts1)
        ebase = c * half + s * per_t
        pltpu.sync_copy(idx_hbm.at[pl.ds(blk, 1)], idx_v)
        pltpu.async_copy(tr_hbm.at[pl.ds(ebase, C)], tb[0], ts[0])

        def pair(g, carry):
            for b in (0, 1):
                i = 2 * g + b
                nb = 1 - b
                pltpu.make_async_copy(
                    tr_hbm.at[pl.ds(ebase + i * C, C)], tb[b], ts[b]).wait()

                @pl.when(i + 1 < iters)
                def _():
                    pltpu.async_copy(
                        tr_hbm.at[pl.ds(ebase + (i + 1) * C, C)],
                        tb[nb], ts[nb])

                pltpu.sync_copy(tb[b], acct.at[idx_v.at[0, i]], add=True)
            return carry

        lax.fori_loop(0, iters // 2, pair, 0)
        plsc.subcore_barrier()
        pltpu.sync_copy(acct.at[nslice], out_hbm.at[c].at[nslice])

    return scattr_k(tr16, row3, z16)


def _node_body(aggc_ref, h_ref, aggr_ref, aggt_ref, coord_ref, vel_ref,
               wn1ot_ref, wn1ht_ref, wn1at_ref, bn1_ref, wn2t_ref, bn2_ref,
               wv1t_ref, bv1_ref, wv2t_ref, bv2_ref,
               hout_ref, cout_ref, vout_ref):
    h = h_ref[...]
    aggt = aggt_ref[0] + aggt_ref[1]
    cnt = jnp.maximum(aggt[:, 3:4], 1.0)
    f = aggt[:, 0:3] / cnt
    hv = jnp.maximum(
        jnp.dot(h, wv1t_ref[...], preferred_element_type=F32) + bv1_ref[...],
        0.0)
    phi = jnp.dot(hv, wv2t_ref[...], preferred_element_type=F32) + bv2_ref[...]
    vel_new = phi * vel_ref[...] + f
    vout_ref[...] = vel_new
    cout_ref[...] = coord_ref[...] + vel_new
    pre = (jnp.dot(aggc_ref[...], wn1ot_ref[...], preferred_element_type=F32)
           + jnp.dot(h, wn1ht_ref[...], preferred_element_type=F32)
           + jnp.dot(aggr_ref[...], wn1at_ref[...], preferred_element_type=F32)
           + bn1_ref[...])
    hn = jnp.dot(jnp.maximum(pre, 0.0), wn2t_ref[...],
                 preferred_element_type=F32) + bn2_ref[...]
    hout_ref[...] = h + hn


def _node_update(aggc, h, aggr, aggt2, coord, vel, wn1ot, wn1ht, wn1at, bn1,
                 wn2t, bn2, wv1t, bv1, wv2t, bv2, N, BN):
    nb = N // BN
    wspec = lambda shape: pl.BlockSpec(shape, lambda i: tuple(0 for _ in shape))
    return pl.pallas_call(
        _node_body,
        grid=(nb,),
        in_specs=[
            pl.BlockSpec((BN, 128), lambda i: (i, 0)),
            pl.BlockSpec((BN, 128), lambda i: (i, 0)),
            pl.BlockSpec((BN, 128), lambda i: (i, 0)),
            pl.BlockSpec((2, BN, 16), lambda i: (0, i, 0)),
            pl.BlockSpec((BN, 3), lambda i: (i, 0)),
            pl.BlockSpec((BN, 3), lambda i: (i, 0)),
            wspec((128, 128)),
            wspec((128, 128)),
            wspec((128, 128)),
            wspec((1, 128)),
            wspec((128, 128)),
            wspec((1, 128)),
            wspec((128, 128)),
            wspec((1, 128)),
            wspec((128, 1)),
            wspec((1, 1)),
        ],
        out_specs=[
            pl.BlockSpec((BN, 128), lambda i: (i, 0)),
            pl.BlockSpec((BN, 3), lambda i: (i, 0)),
            pl.BlockSpec((BN, 3), lambda i: (i, 0)),
        ],
        out_shape=[
            jax.ShapeDtypeStruct((N, 128), F32),
            jax.ShapeDtypeStruct((N, 3), F32),
            jax.ShapeDtypeStruct((N, 3), F32),
        ],
    )(aggc, h, aggr, aggt2, coord, vel, wn1ot, wn1ht, wn1at, bn1, wn2t, bn2,
      wv1t, bv1, wv2t, bv2)


def kernel(h, edge_index, coord, vel, edge_attr, We1, be1, We2, be2,
           Wn1, bn1, Wn2, bn2, Wc1, bc1, Wc2, Wv1, bv1, Wv2, bv2):
    N, D = h.shape
    E = edge_index.shape[1]
    BN = 2000
    BE = 2048
    EP = ((E + 8191) // 8192) * 8192
    PAD = EP - E
    NA = N + 16

    row = edge_index[0].astype(jnp.int32)
    col = edge_index[1].astype(jnp.int32)
    rowp = jnp.pad(row, (0, PAD))
    colp = jnp.pad(col, (0, PAD))
    rows_s = jnp.pad(row, (0, PAD), constant_values=N)
    cols_s = jnp.pad(col, (0, PAD), constant_values=N)

    w_stk = jnp.stack([We1[:, :D].T, We1[:, D:2 * D].T])
    b_stk = jnp.stack([jnp.zeros((1, 128), F32), be1.reshape(1, 128)])
    we1r = We1[:, 2 * D:2 * D + 1].T
    we1et = We1[:, 2 * D + 1:].T
    c16 = jnp.pad(coord, ((0, 0), (0, 13)))
    ea_p = jnp.pad(edge_attr, ((0, PAD), (0, 0)))

    table2 = _precompute_tables(h, w_stk, b_stk, N, BN)

    cat_idx = jnp.concatenate([rowp, colp + N]).reshape(32, -1, C)
    cat_rc = jnp.concatenate([rowp, colp]).reshape(32, -1, C)
    gp = _ring_gather(table2, cat_idx, 2 * EP, 128, True)
    g16 = _ring_gather(c16, cat_rc, 2 * EP, 16, False)

    feat, tr16 = _edge_mlp(gp, g16, ea_p, we1r, we1et, We2.T,
                           be2.reshape(1, 128), Wc1.T, bc1.reshape(1, 128),
                           Wc2.T, EP, BE)

    z128 = jnp.zeros((NA, 128), F32)
    z16 = jnp.zeros((NA, 16), F32)
    cat_s = jnp.concatenate([rows_s, cols_s]).reshape(32, -1, C)
    aggr, aggc = _segment_feat(feat, cat_s, z128, NA, EP)
    aggt2 = _segment_tr(tr16, rows_s.reshape(32, -1, C), z16, NA, EP)

    h_new, coord_new, vel_new = _node_update(
        aggc, h, aggr, aggt2, coord, vel,
        Wn1[:, :128].T, Wn1[:, 128:256].T, Wn1[:, 256:].T,
        bn1.reshape(1, 128), Wn2.T, bn2.reshape(1, 128),
        Wv1.T, bv1.reshape(1, 128), Wv2.T, bv2.reshape(1, 1), N, BN)
    return h_new, coord_new, vel_new


# speedup vs baseline: 3.2604x; 1.0061x over previous
import functools

import jax
import jax.numpy as jnp
from jax import lax
from jax.experimental import pallas as pl
from jax.experimental.pallas import tpu as pltpu
from jax.experimental.pallas import tpu_sc as plsc

F32 = jnp.float32
C = 128


def _pre_body(h_ref, w_ref, b_ref, out_ref):
    out_ref[...] = jnp.dot(h_ref[...], w_ref[0],
                           preferred_element_type=F32) + b_ref[0]


def _precompute_tables(h, w_stk, b_stk, N, BN):
    nb = N // BN
    return pl.pallas_call(
        _pre_body,
        grid=(2, nb),
        in_specs=[
            pl.BlockSpec((BN, 128), lambda j, i: (i, 0)),
            pl.BlockSpec((1, 128, 128), lambda j, i: (j, 0, 0)),
            pl.BlockSpec((1, 1, 128), lambda j, i: (j, 0, 0)),
        ],
        out_specs=pl.BlockSpec((BN, 128), lambda j, i: (j * nb + i, 0)),
        out_shape=jax.ShapeDtypeStruct((2 * N, 128), F32),
    )(h, w_stk, b_stk)


def _ring_gather(table, idx3d, E2, W, tc_tiling):
    NW = 32
    per_w = E2 // NW
    iters = per_w // C
    mesh = plsc.VectorSubcoreMesh(core_axis_name="c", subcore_axis_name="s")

    @functools.partial(
        pl.kernel,
        mesh=mesh,
        out_type=jax.ShapeDtypeStruct((E2, W), F32),
        scratch_types=[
            pltpu.VMEM((1, iters, C), jnp.int32),
            pltpu.VMEM((C, W), F32),
            pltpu.VMEM((C, W), F32),
            pltpu.SemaphoreType.DMA,
            pltpu.SemaphoreType.DMA,
            pltpu.SemaphoreType.DMA,
            pltpu.SemaphoreType.DMA,
        ],
        compiler_params=pltpu.CompilerParams(use_tc_tiling_on_sc=tc_tiling),
    )
    def gather_k(table_hbm, idx_hbm, out_hbm, idx_v, r0, r1, g0, g1, w0, w1):
        wid = lax.axis_index("c") * 16 + lax.axis_index("s")
        base = wid * per_w
        rows = (r0, r1)
        gs = (g0, g1)
        ws = (w0, w1)
        pltpu.sync_copy(idx_hbm.at[pl.ds(wid, 1)], idx_v)
        pltpu.async_copy(table_hbm.at[idx_v.at[0, 0]], rows[0], gs[0])

        def pair(g, carry):
            for b in (0, 1):
                i = 2 * g + b
                nb = 1 - b
                pltpu.make_async_copy(
                    table_hbm.at[idx_v.at[0, i]], rows[b], gs[b]).wait()

                @pl.when(i >= 1)
                def _():
                    pltpu.make_async_copy(
                        rows[nb], out_hbm.at[pl.ds(base + (i - 1) * C, C)],
                        ws[nb]).wait()

                @pl.when(i + 1 < iters)
                def _():
                    pltpu.async_copy(
                        table_hbm.at[idx_v.at[0, i + 1]], rows[nb], gs[nb])

                pltpu.async_copy(
                    rows[b], out_hbm.at[pl.ds(base + i * C, C)], ws[b])
            return carry

        lax.fori_loop(0, iters // 2, pair, 0)
        pltpu.make_async_copy(
            rows[1], out_hbm.at[pl.ds(base + (iters - 1) * C, C)], ws[1]).wait()

    return gather_k(table, idx3d)


def _edge_body(gpr_ref, gpc_ref, gcr_ref, gcc_ref, ea_ref, we1r_ref,
               we1et_ref, we2t_ref, be2_ref, wc1t_ref, bc1_ref, wc2t_ref,
               feat_ref, tr_ref):
    a = gpr_ref[...] + gpc_ref[...]
    cd = gcr_ref[...] - gcc_ref[...]
    radial = jnp.sum(cd * cd, axis=1, keepdims=True)
    ef = a + radial * we1r_ref[...] + jnp.dot(
        ea_ref[...], we1et_ref[...], preferred_element_type=F32)
    ef = jnp.maximum(ef, 0.0)
    feat = jnp.maximum(
        jnp.dot(ef, we2t_ref[...], preferred_element_type=F32) + be2_ref[...],
        0.0)
    feat_ref[...] = feat
    cm = jnp.maximum(
        jnp.dot(feat, wc1t_ref[...], preferred_element_type=F32) + bc1_ref[...],
        0.0)
    m = jnp.dot(cm, wc2t_ref[...], preferred_element_type=F32)
    tr = jnp.clip(cd * m, -100.0, 100.0)
    one3 = (lax.broadcasted_iota(jnp.int32, (1, 16), 1) == 3).astype(F32)
    tr_ref[...] = tr + one3


def _edge_mlp(gp, g16, ea, we1r, we1et, we2t, be2, wc1t, bc1, wc2t, EP, BE):
    nb = EP // BE
    wspec = lambda shape: pl.BlockSpec(shape, lambda i: tuple(0 for _ in shape))
    return pl.pallas_call(
        _edge_body,
        grid=(nb,),
        in_specs=[
            pl.BlockSpec((BE, 128), lambda i: (i, 0)),
            pl.BlockSpec((BE, 128), lambda i: (i + nb, 0)),
            pl.BlockSpec((BE, 16), lambda i: (i, 0)),
            pl.BlockSpec((BE, 16), lambda i: (i + nb, 0)),
            pl.BlockSpec((BE, 16), lambda i: (i, 0)),
            wspec((1, 128)),
            wspec((16, 128)),
            wspec((128, 128)),
            wspec((1, 128)),
            wspec((128, 128)),
            wspec((1, 128)),
            wspec((128, 1)),
        ],
        out_specs=[
            pl.BlockSpec((BE, 128), lambda i: (i, 0)),
            pl.BlockSpec((BE, 16), lambda i: (i, 0)),
        ],
        out_shape=[
            jax.ShapeDtypeStruct((EP, 128), F32),
            jax.ShapeDtypeStruct((EP, 16), F32),
        ],
    )(gp, gp, g16, g16, ea, we1r, we1et, we2t, be2, wc1t, bc1, wc2t)


def _segment_feat(feat, cat_rc3, z128, NA, EP):
    NT = 16
    per_t = EP // NT
    iters = per_t // C
    HI = iters // 2
    mesh = plsc.VectorSubcoreMesh(core_axis_name="c", subcore_axis_name="s")

    @functools.partial(
        pl.kernel,
        mesh=mesh,
        out_type=(
            jax.ShapeDtypeStruct((NA, 128), F32),
            jax.ShapeDtypeStruct((NA, 128), F32),
        ),
        scratch_types=[
            pltpu.VMEM((1, HI, C), jnp.int32),
            pltpu.VMEM((C, 128), F32),
            pltpu.VMEM((C, 128), F32),
            pltpu.SemaphoreType.DMA,
            pltpu.SemaphoreType.DMA,
            pltpu.SemaphoreType.DMA,
            pltpu.VMEM_SHARED((NA, 128), F32),
        ],
    )
    def scatter_k(feat_hbm, idx_hbm, z_hbm, outr_hbm, outc_hbm,
                  idx_v, f0, f1, fs0, fs1, ssem, accf):
        c = lax.axis_index("c")
        s = lax.axis_index("s")
        blk = c * 16 + s
        base = s * 640
        tail = NA - 15 * 640

        @pl.when(s < 15)
        def _():
            sl = pl.ds(base, 640)
            pltpu.sync_copy(z_hbm.at[sl], accf.at[sl])

        @pl.when(s == 15)
        def _():
            sl = pl.ds(base, tail)
            pltpu.sync_copy(z_hbm.at[sl], accf.at[sl])

        plsc.subcore_barrier()

        fb = (f0, f1)
        fs = (fs0, fs1)
        ebase = s * per_t

        for h in (0, 1):
            pltpu.sync_copy(idx_hbm.at[pl.ds(blk, 1), pl.ds(h * HI, HI)],
                            idx_v)
            hbase = ebase + h * HI * C
            pltpu.async_copy(feat_hbm.at[pl.ds(hbase, C)], fb[0], fs[0])

            def pair(g, carry):
                for b in (0, 1):
                    i = 2 * g + b
                    nb = 1 - b
                    pltpu.make_async_copy(
                        feat_hbm.at[pl.ds(hbase + i * C, C)],
                        fb[b], fs[b]).wait()

                    @pl.when(i + 1 < HI)
                    def _():
                        pltpu.async_copy(
                            feat_hbm.at[pl.ds(hbase + (i + 1) * C, C)],
                            fb[nb], fs[nb])

                    pltpu.sync_copy(fb[b], accf.at[idx_v.at[0, i]], add=True)
                return carry

            lax.fori_loop(0, HI // 2, pair, 0)

        plsc.subcore_barrier()

        @pl.when((c == 0) & (s < 15))
        def _():
            sl = pl.ds(base, 640)
            pltpu.sync_copy(accf.at[sl], outr_hbm.at[sl])

        @pl.when((c == 0) & (s == 15))
        def _():
            sl = pl.ds(base, tail)
            pltpu.sync_copy(accf.at[sl], outr_hbm.at[sl])

        @pl.when((c == 1) & (s < 15))
        def _():
            sl = pl.ds(base, 640)
            pltpu.sync_copy(accf.at[sl], outc_hbm.at[sl])

        @pl.when((c == 1) & (s == 15))
        def _():
            sl = pl.ds(base, tail)
            pltpu.sync_copy(accf.at[sl], outc_hbm.at[sl])

    return scatter_k(feat, cat_rc3, z128)


def _segment_tr(tr16, row3, z16, NA, EP):
    NT = 16
    half = EP // 2
    per_t = half // NT
    iters = per_t // C
    rows_t = NA // NT
    mesh = plsc.VectorSubcoreMesh(core_axis_name="c", subcore_axis_name="s")

    @functools.partial(
        pl.kernel,
        mesh=mesh,
        out_type=jax.ShapeDtypeStruct((2, NA, 16), F32),
        scratch_types=[
            pltpu.VMEM((1, iters, C), jnp.int32),
            pltpu.VMEM((C, 16), F32),
            pltpu.VMEM((C, 16), F32),
            pltpu.SemaphoreType.DMA,
            pltpu.SemaphoreType.DMA,
            pltpu.SemaphoreType.DMA,
            pltpu.VMEM_SHARED((NA, 16), F32),
        ],
        compiler_params=pltpu.CompilerParams(use_tc_tiling_on_sc=False),
    )
    def scattr_k(tr_hbm, idx_hbm, z_hbm, out_hbm,
                 idx_v, t0, t1, ts0, ts1, ssem, acct):
        c = lax.axis_index("c")
        s = lax.axis_index("s")
        blk = c * 16 + s
        nslice = pl.ds(s * rows_t, rows_t)
        pltpu.sync_copy(z_hbm.at[nslice], acct.at[nslice])
        plsc.subcore_barrier()

        tb = (t0, t1)
        ts = (ts0, ts1)
        ebase = c * half + s * per_t
        pltpu.sync_copy(idx_hbm.at[pl.ds(blk, 1)], idx_v)
        pltpu.async_copy(tr_hbm.at[pl.ds(ebase, C)], tb[0], ts[0])

        def pair(g, carry):
            for b in (0, 1):
                i = 2 * g + b
                nb = 1 - b
                pltpu.make_async_copy(
                    tr_hbm.at[pl.ds(ebase + i * C, C)], tb[b], ts[b]).wait()

                @pl.when(i + 1 < iters)
                def _():
                    pltpu.async_copy(
                        tr_hbm.at[pl.ds(ebase + (i + 1) * C, C)],
                        tb[nb], ts[nb])

                pltpu.sync_copy(tb[b], acct.at[idx_v.at[0, i]], add=True)
            return carry

        lax.fori_loop(0, iters // 2, pair, 0)
        plsc.subcore_barrier()
        pltpu.sync_copy(acct.at[nslice], out_hbm.at[c].at[nslice])

    return scattr_k(tr16, row3, z16)


def _node_body(aggc_ref, h_ref, aggr_ref, aggt_ref, coord_ref, vel_ref,
               wn1ot_ref, wn1ht_ref, wn1at_ref, bn1_ref, wn2t_ref, bn2_ref,
               wv1t_ref, bv1_ref, wv2t_ref, bv2_ref,
               hout_ref, cout_ref, vout_ref):
    h = h_ref[...]
    aggt = aggt_ref[0] + aggt_ref[1]
    cnt = jnp.maximum(aggt[:, 3:4], 1.0)
    f = aggt[:, 0:3] / cnt
    hv = jnp.maximum(
        jnp.dot(h, wv1t_ref[...], preferred_element_type=F32) + bv1_ref[...],
        0.0)
    phi = jnp.dot(hv, wv2t_ref[...], preferred_element_type=F32) + bv2_ref[...]
    vel_new = phi * vel_ref[...] + f
    vout_ref[...] = vel_new
    cout_ref[...] = coord_ref[...] + vel_new
    pre = (jnp.dot(aggc_ref[...], wn1ot_ref[...], preferred_element_type=F32)
           + jnp.dot(h, wn1ht_ref[...], preferred_element_type=F32)
           + jnp.dot(aggr_ref[...], wn1at_ref[...], preferred_element_type=F32)
           + bn1_ref[...])
    hn = jnp.dot(jnp.maximum(pre, 0.0), wn2t_ref[...],
                 preferred_element_type=F32) + bn2_ref[...]
    hout_ref[...] = h + hn


def _node_update(aggc, h, aggr, aggt2, coord, vel, wn1ot, wn1ht, wn1at, bn1,
                 wn2t, bn2, wv1t, bv1, wv2t, bv2, N, BN):
    nb = N // BN
    wspec = lambda shape: pl.BlockSpec(shape, lambda i: tuple(0 for _ in shape))
    return pl.pallas_call(
        _node_body,
        grid=(nb,),
        in_specs=[
            pl.BlockSpec((BN, 128), lambda i: (i, 0)),
            pl.BlockSpec((BN, 128), lambda i: (i, 0)),
            pl.BlockSpec((BN, 128), lambda i: (i, 0)),
            pl.BlockSpec((2, BN, 16), lambda i: (0, i, 0)),
            pl.BlockSpec((BN, 3), lambda i: (i, 0)),
            pl.BlockSpec((BN, 3), lambda i: (i, 0)),
            wspec((128, 128)),
            wspec((128, 128)),
            wspec((128, 128)),
            wspec((1, 128)),
            wspec((128, 128)),
            wspec((1, 128)),
            wspec((128, 128)),
            wspec((1, 128)),
            wspec((128, 1)),
            wspec((1, 1)),
        ],
        out_specs=[
            pl.BlockSpec((BN, 128), lambda i: (i, 0)),
            pl.BlockSpec((BN, 3), lambda i: (i, 0)),
            pl.BlockSpec((BN, 3), lambda i: (i, 0)),
        ],
        out_shape=[
            jax.ShapeDtypeStruct((N, 128), F32),
            jax.ShapeDtypeStruct((N, 3), F32),
            jax.ShapeDtypeStruct((N, 3), F32),
        ],
    )(aggc, h, aggr, aggt2, coord, vel, wn1ot, wn1ht, wn1at, bn1, wn2t, bn2,
      wv1t, bv1, wv2t, bv2)


def kernel(h, edge_index, coord, vel, edge_attr, We1, be1, We2, be2,
           Wn1, bn1, Wn2, bn2, Wc1, bc1, Wc2, Wv1, bv1, Wv2, bv2):
    N, D = h.shape
    E = edge_index.shape[1]
    BN = 2000
    BE = 2048
    EP = ((E + 8191) // 8192) * 8192
    PAD = EP - E
    NA = N + 16

    row = edge_index[0].astype(jnp.int32)
    col = edge_index[1].astype(jnp.int32)
    rowp = jnp.pad(row, (0, PAD))
    colp = jnp.pad(col, (0, PAD))
    rows_s = jnp.pad(row, (0, PAD), constant_values=N)
    cols_s = jnp.pad(col, (0, PAD), constant_values=N)

    w_stk = jnp.stack([We1[:, :D].T, We1[:, D:2 * D].T])
    b_stk = jnp.stack([jnp.zeros((1, 128), F32), be1.reshape(1, 128)])
    we1r = We1[:, 2 * D:2 * D + 1].T
    we1et = We1[:, 2 * D + 1:].T
    c16 = jnp.pad(coord, ((0, 0), (0, 13)))
    ea_p = jnp.pad(edge_attr, ((0, PAD), (0, 0)))

    table2 = _precompute_tables(h, w_stk, b_stk, N, BN)

    cat_idx = jnp.concatenate([rowp, colp + N]).reshape(32, -1, C)
    cat_rc = jnp.concatenate([rowp, colp]).reshape(32, -1, C)
    gp = _ring_gather(table2, cat_idx, 2 * EP, 128, True)
    g16 = _ring_gather(c16, cat_rc, 2 * EP, 16, False)

    feat, tr16 = _edge_mlp(gp, g16, ea_p, we1r, we1et, We2.T,
                           be2.reshape(1, 128), Wc1.T, bc1.reshape(1, 128),
                           Wc2.T, EP, BE)

    z128 = jnp.zeros((NA, 128), F32)
    z16 = jnp.zeros((NA, 16), F32)
    cat_s = jnp.concatenate([rows_s, cols_s]).reshape(32, -1, C)
    aggt2 = _segment_tr(tr16, rows_s.reshape(32, -1, C), z16, NA, EP)
    aggr, aggc = _segment_feat(feat, cat_s, z128, NA, EP)

    h_new, coord_new, vel_new = _node_update(
        aggc, h, aggr, aggt2, coord, vel,
        Wn1[:, :128].T, Wn1[:, 128:256].T, Wn1[:, 256:].T,
        bn1.reshape(1, 128), Wn2.T, bn2.reshape(1, 128),
        Wv1.T, bv1.reshape(1, 128), Wv2.T, bv2.reshape(1, 1), N, BN)
    return h_new, coord_new, vel_new


# speedup vs baseline: 3.3532x; 1.0285x over previous
import functools

import jax
import jax.numpy as jnp
from jax import lax
from jax.experimental import pallas as pl
from jax.experimental.pallas import tpu as pltpu
from jax.experimental.pallas import tpu_sc as plsc

F32 = jnp.float32
C = 128


def _pre_body(h_ref, w_ref, b_ref, out_ref):
    out_ref[...] = jnp.dot(h_ref[...], w_ref[0],
                           preferred_element_type=F32) + b_ref[0]


def _precompute_tables(h, w_stk, b_stk, N, BN):
    nb = N // BN
    return pl.pallas_call(
        _pre_body,
        grid=(2, nb),
        in_specs=[
            pl.BlockSpec((BN, 128), lambda j, i: (i, 0)),
            pl.BlockSpec((1, 128, 128), lambda j, i: (j, 0, 0)),
            pl.BlockSpec((1, 1, 128), lambda j, i: (j, 0, 0)),
        ],
        out_specs=pl.BlockSpec((BN, 128), lambda j, i: (j * nb + i, 0)),
        out_shape=jax.ShapeDtypeStruct((2 * N, 128), F32),
    )(h, w_stk, b_stk)


def _ring_gather(table, idx3d, E2, W, tc_tiling):
    NW = 32
    per_w = E2 // NW
    iters = per_w // C
    mesh = plsc.VectorSubcoreMesh(core_axis_name="c", subcore_axis_name="s")

    @functools.partial(
        pl.kernel,
        mesh=mesh,
        out_type=jax.ShapeDtypeStruct((E2, W), F32),
        scratch_types=[
            pltpu.VMEM((1, iters, C), jnp.int32),
            pltpu.VMEM((C, W), F32),
            pltpu.VMEM((C, W), F32),
            pltpu.VMEM((C, W), F32),
            pltpu.VMEM((C, W), F32),
            pltpu.SemaphoreType.DMA,
            pltpu.SemaphoreType.DMA,
            pltpu.SemaphoreType.DMA,
            pltpu.SemaphoreType.DMA,
            pltpu.SemaphoreType.DMA,
            pltpu.SemaphoreType.DMA,
            pltpu.SemaphoreType.DMA,
            pltpu.SemaphoreType.DMA,
        ],
        compiler_params=pltpu.CompilerParams(use_tc_tiling_on_sc=tc_tiling),
    )
    def gather_k(table_hbm, idx_hbm, out_hbm, idx_v,
                 r0, r1, r2, r3, g0, g1, g2, g3, w0, w1, w2, w3):
        wid = lax.axis_index("c") * 16 + lax.axis_index("s")
        base = wid * per_w
        rows = (r0, r1, r2, r3)
        gs = (g0, g1, g2, g3)
        ws = (w0, w1, w2, w3)
        pltpu.sync_copy(idx_hbm.at[pl.ds(wid, 1)], idx_v)
        for b in (0, 1, 2):
            pltpu.async_copy(table_hbm.at[idx_v.at[0, b]], rows[b], gs[b])

        def quad(g, carry):
            for b in (0, 1, 2, 3):
                i = 4 * g + b
                nb = (b + 3) % 4
                pltpu.make_async_copy(
                    table_hbm.at[idx_v.at[0, i]], rows[b], gs[b]).wait()

                @pl.when((i >= 1) & (i + 3 < iters))
                def _():
                    pltpu.make_async_copy(
                        rows[nb], out_hbm.at[pl.ds(base + (i - 1) * C, C)],
                        ws[nb]).wait()

                @pl.when(i + 3 < iters)
                def _():
                    pltpu.async_copy(
                        table_hbm.at[idx_v.at[0, i + 3]], rows[nb], gs[nb])

                pltpu.async_copy(
                    rows[b], out_hbm.at[pl.ds(base + i * C, C)], ws[b])
            return carry

        lax.fori_loop(0, iters // 4, quad, 0)
        for k in range(4):
            i = iters - 4 + k
            pltpu.make_async_copy(
                rows[i % 4], out_hbm.at[pl.ds(base + i * C, C)],
                ws[i % 4]).wait()

    return gather_k(table, idx3d)


def _edge_body(gpr_ref, gpc_ref, gcr_ref, gcc_ref, ea_ref, we1r_ref,
               we1et_ref, we2t_ref, be2_ref, wc1t_ref, bc1_ref, wc2t_ref,
               feat_ref, tr_ref):
    a = gpr_ref[...] + gpc_ref[...]
    cd = gcr_ref[...] - gcc_ref[...]
    radial = jnp.sum(cd * cd, axis=1, keepdims=True)
    ef = a + radial * we1r_ref[...] + jnp.dot(
        ea_ref[...], we1et_ref[...], preferred_element_type=F32)
    ef = jnp.maximum(ef, 0.0)
    feat = jnp.maximum(
        jnp.dot(ef, we2t_ref[...], preferred_element_type=F32) + be2_ref[...],
        0.0)
    feat_ref[...] = feat
    cm = jnp.maximum(
        jnp.dot(feat, wc1t_ref[...], preferred_element_type=F32) + bc1_ref[...],
        0.0)
    m = jnp.dot(cm, wc2t_ref[...], preferred_element_type=F32)
    tr = jnp.clip(cd * m, -100.0, 100.0)
    one3 = (lax.broadcasted_iota(jnp.int32, (1, 16), 1) == 3).astype(F32)
    tr_ref[...] = tr + one3


def _edge_mlp(gp, g16, ea, we1r, we1et, we2t, be2, wc1t, bc1, wc2t, EP, BE):
    nb = EP // BE
    wspec = lambda shape: pl.BlockSpec(shape, lambda i: tuple(0 for _ in shape))
    return pl.pallas_call(
        _edge_body,
        grid=(nb,),
        in_specs=[
            pl.BlockSpec((BE, 128), lambda i: (i, 0)),
            pl.BlockSpec((BE, 128), lambda i: (i + nb, 0)),
            pl.BlockSpec((BE, 16), lambda i: (i, 0)),
            pl.BlockSpec((BE, 16), lambda i: (i + nb, 0)),
            pl.BlockSpec((BE, 16), lambda i: (i, 0)),
            wspec((1, 128)),
            wspec((16, 128)),
            wspec((128, 128)),
            wspec((1, 128)),
            wspec((128, 128)),
            wspec((1, 128)),
            wspec((128, 1)),
        ],
        out_specs=[
            pl.BlockSpec((BE, 128), lambda i: (i, 0)),
            pl.BlockSpec((BE, 16), lambda i: (i, 0)),
        ],
        out_shape=[
            jax.ShapeDtypeStruct((EP, 128), F32),
            jax.ShapeDtypeStruct((EP, 16), F32),
        ],
    )(gp, gp, g16, g16, ea, we1r, we1et, we2t, be2, wc1t, bc1, wc2t)


def _segment_feat(feat, cat_rc3, z128, NA, EP):
    NT = 16
    per_t = EP // NT
    iters = per_t // C
    HI = iters // 2
    mesh = plsc.VectorSubcoreMesh(core_axis_name="c", subcore_axis_name="s")

    @functools.partial(
        pl.kernel,
        mesh=mesh,
        out_type=(
            jax.ShapeDtypeStruct((NA, 128), F32),
            jax.ShapeDtypeStruct((NA, 128), F32),
        ),
        scratch_types=[
            pltpu.VMEM((1, HI, C), jnp.int32),
            pltpu.VMEM((C, 128), F32),
            pltpu.VMEM((C, 128), F32),
            pltpu.SemaphoreType.DMA,
            pltpu.SemaphoreType.DMA,
            pltpu.SemaphoreType.DMA,
            pltpu.VMEM_SHARED((NA, 128), F32),
        ],
    )
    def scatter_k(feat_hbm, idx_hbm, z_hbm, outr_hbm, outc_hbm,
                  idx_v, f0, f1, fs0, fs1, ssem, accf):
        c = lax.axis_index("c")
        s = lax.axis_index("s")
        blk = c * 16 + s
        base = s * 640
        tail = NA - 15 * 640

        @pl.when(s < 15)
        def _():
            sl = pl.ds(base, 640)
            pltpu.sync_copy(z_hbm.at[sl], accf.at[sl])

        @pl.when(s == 15)
        def _():
            sl = pl.ds(base, tail)
            pltpu.sync_copy(z_hbm.at[sl], accf.at[sl])

        plsc.subcore_barrier()

        fb = (f0, f1)
        fs = (fs0, fs1)
        ebase = s * per_t

        for h in (0, 1):
            pltpu.sync_copy(idx_hbm.at[pl.ds(blk, 1), pl.ds(h * HI, HI)],
                            idx_v)
            hbase = ebase + h * HI * C
            pltpu.async_copy(feat_hbm.at[pl.ds(hbase, C)], fb[0], fs[0])

            def pair(g, carry):
                for b in (0, 1):
                    i = 2 * g + b
                    nb = 1 - b
                    pltpu.make_async_copy(
                        feat_hbm.at[pl.ds(hbase + i * C, C)],
                        fb[b], fs[b]).wait()

                    @pl.when(i + 1 < HI)
                    def _():
                        pltpu.async_copy(
                            feat_hbm.at[pl.ds(hbase + (i + 1) * C, C)],
                            fb[nb], fs[nb])

                    pltpu.sync_copy(fb[b], accf.at[idx_v.at[0, i]], add=True)
                return carry

            lax.fori_loop(0, HI // 2, pair, 0)

        plsc.subcore_barrier()

        @pl.when((c == 0) & (s < 15))
        def _():
            sl = pl.ds(base, 640)
            pltpu.sync_copy(accf.at[sl], outr_hbm.at[sl])

        @pl.when((c == 0) & (s == 15))
        def _():
            sl = pl.ds(base, tail)
            pltpu.sync_copy(accf.at[sl], outr_hbm.at[sl])

        @pl.when((c == 1) & (s < 15))
        def _():
            sl = pl.ds(base, 640)
            pltpu.sync_copy(accf.at[sl], outc_hbm.at[sl])

        @pl.when((c == 1) & (s == 15))
        def _():
            sl = pl.ds(base, tail)
            pltpu.sync_copy(accf.at[sl], outc_hbm.at[sl])

    return scatter_k(feat, cat_rc3, z128)


def _segment_tr(tr16, row3, z16, NA, EP):
    NT = 16
    half = EP // 2
    per_t = half // NT
    iters = per_t // C
    rows_t = NA // NT
    mesh = plsc.VectorSubcoreMesh(core_axis_name="c", subcore_axis_name="s")

    @functools.partial(
        pl.kernel,
        mesh=mesh,
        out_type=jax.ShapeDtypeStruct((2, NA, 16), F32),
        scratch_types=[
            pltpu.VMEM((1, iters, C), jnp.int32),
            pltpu.VMEM((C, 16), F32),
            pltpu.VMEM((C, 16), F32),
            pltpu.SemaphoreType.DMA,
            pltpu.SemaphoreType.DMA,
            pltpu.SemaphoreType.DMA,
            pltpu.VMEM_SHARED((NA, 16), F32),
        ],
        compiler_params=pltpu.CompilerParams(use_tc_tiling_on_sc=False),
    )
    def scattr_k(tr_hbm, idx_hbm, z_hbm, out_hbm,
                 idx_v, t0, t1, ts0, ts1, ssem, acct):
        c = lax.axis_index("c")
        s = lax.axis_index("s")
        blk = c * 16 + s
        nslice = pl.ds(s * rows_t, rows_t)
        pltpu.sync_copy(z_hbm.at[nslice], acct.at[nslice])
        plsc.subcore_barrier()

        tb = (t0, t1)
        ts = (ts0, ts1)
        ebase = c * half + s * per_t
        pltpu.sync_copy(idx_hbm.at[pl.ds(blk, 1)], idx_v)
        pltpu.async_copy(tr_hbm.at[pl.ds(ebase, C)], tb[0], ts[0])

        def pair(g, carry):
            for b in (0, 1):
                i = 2 * g + b
                nb = 1 - b
                pltpu.make_async_copy(
                    tr_hbm.at[pl.ds(ebase + i * C, C)], tb[b], ts[b]).wait()

                @pl.when(i + 1 < iters)
                def _():
                    pltpu.async_copy(
                        tr_hbm.at[pl.ds(ebase + (i + 1) * C, C)],
                        tb[nb], ts[nb])

                pltpu.sync_copy(tb[b], acct.at[idx_v.at[0, i]], add=True)
            return carry

        lax.fori_loop(0, iters // 2, pair, 0)
        plsc.subcore_barrier()
        pltpu.sync_copy(acct.at[nslice], out_hbm.at[c].at[nslice])

    return scattr_k(tr16, row3, z16)


def _node_body(aggc_ref, h_ref, aggr_ref, aggt_ref, coord_ref, vel_ref,
               wn1ot_ref, wn1ht_ref, wn1at_ref, bn1_ref, wn2t_ref, bn2_ref,
               wv1t_ref, bv1_ref, wv2t_ref, bv2_ref,
               hout_ref, cout_ref, vout_ref):
    h = h_ref[...]
    aggt = aggt_ref[0] + aggt_ref[1]
    cnt = jnp.maximum(aggt[:, 3:4], 1.0)
    f = aggt[:, 0:3] / cnt
    hv = jnp.maximum(
        jnp.dot(h, wv1t_ref[...], preferred_element_type=F32) + bv1_ref[...],
        0.0)
    phi = jnp.dot(hv, wv2t_ref[...], preferred_element_type=F32) + bv2_ref[...]
    vel_new = phi * vel_ref[...] + f
    vout_ref[...] = vel_new
    cout_ref[...] = coord_ref[...] + vel_new
    pre = (jnp.dot(aggc_ref[...], wn1ot_ref[...], preferred_element_type=F32)
           + jnp.dot(h, wn1ht_ref[...], preferred_element_type=F32)
           + jnp.dot(aggr_ref[...], wn1at_ref[...], preferred_element_type=F32)
           + bn1_ref[...])
    hn = jnp.dot(jnp.maximum(pre, 0.0), wn2t_ref[...],
                 preferred_element_type=F32) + bn2_ref[...]
    hout_ref[...] = h + hn


def _node_update(aggc, h, aggr, aggt2, coord, vel, wn1ot, wn1ht, wn1at, bn1,
                 wn2t, bn2, wv1t, bv1, wv2t, bv2, N, BN):
    nb = N // BN
    wspec = lambda shape: pl.BlockSpec(shape, lambda i: tuple(0 for _ in shape))
    return pl.pallas_call(
        _node_body,
        grid=(nb,),
        in_specs=[
            pl.BlockSpec((BN, 128), lambda i: (i, 0)),
            pl.BlockSpec((BN, 128), lambda i: (i, 0)),
            pl.BlockSpec((BN, 128), lambda i: (i, 0)),
            pl.BlockSpec((2, BN, 16), lambda i: (0, i, 0)),
            pl.BlockSpec((BN, 3), lambda i: (i, 0)),
            pl.BlockSpec((BN, 3), lambda i: (i, 0)),
            wspec((128, 128)),
            wspec((128, 128)),
            wspec((128, 128)),
            wspec((1, 128)),
            wspec((128, 128)),
            wspec((1, 128)),
            wspec((128, 128)),
            wspec((1, 128)),
            wspec((128, 1)),
            wspec((1, 1)),
        ],
        out_specs=[
            pl.BlockSpec((BN, 128), lambda i: (i, 0)),
            pl.BlockSpec((BN, 3), lambda i: (i, 0)),
            pl.BlockSpec((BN, 3), lambda i: (i, 0)),
        ],
        out_shape=[
            jax.ShapeDtypeStruct((N, 128), F32),
            jax.ShapeDtypeStruct((N, 3), F32),
            jax.ShapeDtypeStruct((N, 3), F32),
        ],
    )(aggc, h, aggr, aggt2, coord, vel, wn1ot, wn1ht, wn1at, bn1, wn2t, bn2,
      wv1t, bv1, wv2t, bv2)


def kernel(h, edge_index, coord, vel, edge_attr, We1, be1, We2, be2,
           Wn1, bn1, Wn2, bn2, Wc1, bc1, Wc2, Wv1, bv1, Wv2, bv2):
    N, D = h.shape
    E = edge_index.shape[1]
    BN = 2000
    BE = 2048
    EP = ((E + 8191) // 8192) * 8192
    PAD = EP - E
    NA = N + 16

    row = edge_index[0].astype(jnp.int32)
    col = edge_index[1].astype(jnp.int32)
    rowp = jnp.pad(row, (0, PAD))
    colp = jnp.pad(col, (0, PAD))
    rows_s = jnp.pad(row, (0, PAD), constant_values=N)
    cols_s = jnp.pad(col, (0, PAD), constant_values=N)

    w_stk = jnp.stack([We1[:, :D].T, We1[:, D:2 * D].T])
    b_stk = jnp.stack([jnp.zeros((1, 128), F32), be1.reshape(1, 128)])
    we1r = We1[:, 2 * D:2 * D + 1].T
    we1et = We1[:, 2 * D + 1:].T
    c16 = jnp.pad(coord, ((0, 0), (0, 13)))
    ea_p = jnp.pad(edge_attr, ((0, PAD), (0, 0)))

    table2 = _precompute_tables(h, w_stk, b_stk, N, BN)

    cat_idx = jnp.concatenate([rowp, colp + N]).reshape(32, -1, C)
    cat_rc = jnp.concatenate([rowp, colp]).reshape(32, -1, C)
    gp = _ring_gather(table2, cat_idx, 2 * EP, 128, True)
    g16 = _ring_gather(c16, cat_rc, 2 * EP, 16, False)

    feat, tr16 = _edge_mlp(gp, g16, ea_p, we1r, we1et, We2.T,
                           be2.reshape(1, 128), Wc1.T, bc1.reshape(1, 128),
                           Wc2.T, EP, BE)

    z128 = jnp.zeros((NA, 128), F32)
    z16 = jnp.zeros((NA, 16), F32)
    cat_s = jnp.concatenate([rows_s, cols_s]).reshape(32, -1, C)
    aggt2 = _segment_tr(tr16, rows_s.reshape(32, -1, C), z16, NA, EP)
    aggr, aggc = _segment_feat(feat, cat_s, z128, NA, EP)

    h_new, coord_new, vel_new = _node_update(
        aggc, h, aggr, aggt2, coord, vel,
        Wn1[:, :128].T, Wn1[:, 128:256].T, Wn1[:, 256:].T,
        bn1.reshape(1, 128), Wn2.T, bn2.reshape(1, 128),
        Wv1.T, bv1.reshape(1, 128), Wv2.T, bv2.reshape(1, 1), N, BN)
    return h_new, coord_new, vel_new


# speedup vs baseline: 3.3712x; 1.0054x over previous
import functools

import jax
import jax.numpy as jnp
from jax import lax
from jax.experimental import pallas as pl
from jax.experimental.pallas import tpu as pltpu
from jax.experimental.pallas import tpu_sc as plsc

F32 = jnp.float32
C = 128


def _pre_body(h_ref, w_ref, b_ref, out_ref):
    out_ref[...] = jnp.dot(h_ref[...], w_ref[0],
                           preferred_element_type=F32) + b_ref[0]


def _precompute_tables(h, w_stk, b_stk, N, BN):
    nb = N // BN
    return pl.pallas_call(
        _pre_body,
        grid=(2, nb),
        in_specs=[
            pl.BlockSpec((BN, 128), lambda j, i: (i, 0)),
            pl.BlockSpec((1, 128, 128), lambda j, i: (j, 0, 0)),
            pl.BlockSpec((1, 1, 128), lambda j, i: (j, 0, 0)),
        ],
        out_specs=pl.BlockSpec((BN, 128), lambda j, i: (j * nb + i, 0)),
        out_shape=jax.ShapeDtypeStruct((2 * N, 128), F32),
    )(h, w_stk, b_stk)


def _ring_gather(table, idx3d, E2, W, tc_tiling):
    NW = 32
    per_w = E2 // NW
    iters = per_w // C
    mesh = plsc.VectorSubcoreMesh(core_axis_name="c", subcore_axis_name="s")

    @functools.partial(
        pl.kernel,
        mesh=mesh,
        out_type=jax.ShapeDtypeStruct((E2, W), F32),
        scratch_types=[
            pltpu.VMEM((1, iters, C), jnp.int32),
            pltpu.VMEM((C, W), F32),
            pltpu.VMEM((C, W), F32),
            pltpu.VMEM((C, W), F32),
            pltpu.VMEM((C, W), F32),
            pltpu.SemaphoreType.DMA,
            pltpu.SemaphoreType.DMA,
            pltpu.SemaphoreType.DMA,
            pltpu.SemaphoreType.DMA,
            pltpu.SemaphoreType.DMA,
            pltpu.SemaphoreType.DMA,
            pltpu.SemaphoreType.DMA,
            pltpu.SemaphoreType.DMA,
        ],
        compiler_params=pltpu.CompilerParams(use_tc_tiling_on_sc=tc_tiling),
    )
    def gather_k(table_hbm, idx_hbm, out_hbm, idx_v,
                 r0, r1, r2, r3, g0, g1, g2, g3, w0, w1, w2, w3):
        wid = lax.axis_index("c") * 16 + lax.axis_index("s")
        base = wid * per_w
        rows = (r0, r1, r2, r3)
        gs = (g0, g1, g2, g3)
        ws = (w0, w1, w2, w3)
        pltpu.sync_copy(idx_hbm.at[pl.ds(wid, 1)], idx_v)
        for b in (0, 1, 2):
            pltpu.async_copy(table_hbm.at[idx_v.at[0, b]], rows[b], gs[b])

        def quad(g, carry):
            for b in (0, 1, 2, 3):
                i = 4 * g + b
                nb = (b + 3) % 4
                pltpu.make_async_copy(
                    table_hbm.at[idx_v.at[0, i]], rows[b], gs[b]).wait()

                @pl.when((i >= 1) & (i + 3 < iters))
                def _():
                    pltpu.make_async_copy(
                        rows[nb], out_hbm.at[pl.ds(base + (i - 1) * C, C)],
                        ws[nb]).wait()

                @pl.when(i + 3 < iters)
                def _():
                    pltpu.async_copy(
                        table_hbm.at[idx_v.at[0, i + 3]], rows[nb], gs[nb])

                pltpu.async_copy(
                    rows[b], out_hbm.at[pl.ds(base + i * C, C)], ws[b])
            return carry

        lax.fori_loop(0, iters // 4, quad, 0)
        for k in range(4):
            i = iters - 4 + k
            pltpu.make_async_copy(
                rows[i % 4], out_hbm.at[pl.ds(base + i * C, C)],
                ws[i % 4]).wait()

    return gather_k(table, idx3d)


def _edge_body(gpr_ref, gpc_ref, gcr_ref, gcc_ref, ea_ref, we1r_ref,
               we1et_ref, we2t_ref, be2_ref, wc1t_ref, bc1_ref, wc2t_ref,
               feat_ref, tr_ref):
    a = gpr_ref[...] + gpc_ref[...]
    cd = gcr_ref[...] - gcc_ref[...]
    radial = jnp.sum(cd * cd, axis=1, keepdims=True)
    ef = a + radial * we1r_ref[...] + jnp.dot(
        ea_ref[...], we1et_ref[...], preferred_element_type=F32)
    ef = jnp.maximum(ef, 0.0)
    feat = jnp.maximum(
        jnp.dot(ef, we2t_ref[...], preferred_element_type=F32) + be2_ref[...],
        0.0)
    feat_ref[...] = feat
    cm = jnp.maximum(
        jnp.dot(feat, wc1t_ref[...], preferred_element_type=F32) + bc1_ref[...],
        0.0)
    m = jnp.dot(cm, wc2t_ref[...], preferred_element_type=F32)
    tr = jnp.clip(cd * m, -100.0, 100.0)
    one3 = (lax.broadcasted_iota(jnp.int32, (1, 16), 1) == 3).astype(F32)
    tr_ref[...] = tr + one3


def _edge_mlp(gp, g16, ea, we1r, we1et, we2t, be2, wc1t, bc1, wc2t, EP, BE):
    nb = EP // BE
    wspec = lambda shape: pl.BlockSpec(shape, lambda i: tuple(0 for _ in shape))
    return pl.pallas_call(
        _edge_body,
        grid=(nb,),
        in_specs=[
            pl.BlockSpec((BE, 128), lambda i: (i, 0)),
            pl.BlockSpec((BE, 128), lambda i: (i + nb, 0)),
            pl.BlockSpec((BE, 16), lambda i: (i, 0)),
            pl.BlockSpec((BE, 16), lambda i: (i + nb, 0)),
            pl.BlockSpec((BE, 16), lambda i: (i, 0)),
            wspec((1, 128)),
            wspec((16, 128)),
            wspec((128, 128)),
            wspec((1, 128)),
            wspec((128, 128)),
            wspec((1, 128)),
            wspec((128, 1)),
        ],
        out_specs=[
            pl.BlockSpec((BE, 128), lambda i: (i, 0)),
            pl.BlockSpec((BE, 16), lambda i: (i, 0)),
        ],
        out_shape=[
            jax.ShapeDtypeStruct((EP, 128), F32),
            jax.ShapeDtypeStruct((EP, 16), F32),
        ],
    )(gp, gp, g16, g16, ea, we1r, we1et, we2t, be2, wc1t, bc1, wc2t)


def _segment_feat(feat, cat_rc3, z128, NA, EP):
    NT = 16
    per_t = EP // NT
    iters = per_t // C
    HI = iters // 2
    mesh = plsc.VectorSubcoreMesh(core_axis_name="c", subcore_axis_name="s")

    @functools.partial(
        pl.kernel,
        mesh=mesh,
        out_type=(
            jax.ShapeDtypeStruct((NA, 128), F32),
            jax.ShapeDtypeStruct((NA, 128), F32),
        ),
        scratch_types=[
            pltpu.VMEM((1, HI, C), jnp.int32),
            pltpu.VMEM((C, 128), F32),
            pltpu.VMEM((C, 128), F32),
            pltpu.SemaphoreType.DMA,
            pltpu.SemaphoreType.DMA,
            pltpu.SemaphoreType.DMA,
            pltpu.VMEM_SHARED((NA, 128), F32),
        ],
    )
    def scatter_k(feat_hbm, idx_hbm, z_hbm, outr_hbm, outc_hbm,
                  idx_v, f0, f1, fs0, fs1, ssem, accf):
        c = lax.axis_index("c")
        s = lax.axis_index("s")
        blk = c * 16 + s
        base = s * 640
        tail = NA - 15 * 640

        @pl.when(s < 15)
        def _():
            sl = pl.ds(base, 640)
            pltpu.sync_copy(z_hbm.at[sl], accf.at[sl])

        @pl.when(s == 15)
        def _():
            sl = pl.ds(base, tail)
            pltpu.sync_copy(z_hbm.at[sl], accf.at[sl])

        plsc.subcore_barrier()

        fb = (f0, f1)
        fs = (fs0, fs1)
        ebase = s * per_t

        for h in (0, 1):
            pltpu.sync_copy(idx_hbm.at[pl.ds(blk, 1), pl.ds(h * HI, HI)],
                            idx_v)
            hbase = ebase + h * HI * C
            pltpu.async_copy(feat_hbm.at[pl.ds(hbase, C)], fb[0], fs[0])

            def pair(g, carry):
                for b in (0, 1):
                    i = 2 * g + b
                    nb = 1 - b
                    pltpu.make_async_copy(
                        feat_hbm.at[pl.ds(hbase + i * C, C)],
                        fb[b], fs[b]).wait()

                    @pl.when(i + 1 < HI)
                    def _():
                        pltpu.async_copy(
                            feat_hbm.at[pl.ds(hbase + (i + 1) * C, C)],
                            fb[nb], fs[nb])

                    pltpu.sync_copy(fb[b], accf.at[idx_v.at[0, i]], add=True)
                return carry

            lax.fori_loop(0, HI // 2, pair, 0)

        plsc.subcore_barrier()

        @pl.when((c == 0) & (s < 15))
        def _():
            sl = pl.ds(base, 640)
            pltpu.sync_copy(accf.at[sl], outr_hbm.at[sl])

        @pl.when((c == 0) & (s == 15))
        def _():
            sl = pl.ds(base, tail)
            pltpu.sync_copy(accf.at[sl], outr_hbm.at[sl])

        @pl.when((c == 1) & (s < 15))
        def _():
            sl = pl.ds(base, 640)
            pltpu.sync_copy(accf.at[sl], outc_hbm.at[sl])

        @pl.when((c == 1) & (s == 15))
        def _():
            sl = pl.ds(base, tail)
            pltpu.sync_copy(accf.at[sl], outc_hbm.at[sl])

    return scatter_k(feat, cat_rc3, z128)


def _segment_tr(tr16, row3, z16, NA, EP):
    NT = 16
    half = EP // 2
    per_t = half // NT
    iters = per_t // C
    rows_t = NA // NT
    mesh = plsc.VectorSubcoreMesh(core_axis_name="c", subcore_axis_name="s")

    @functools.partial(
        pl.kernel,
        mesh=mesh,
        out_type=jax.ShapeDtypeStruct((2, NA, 16), F32),
        scratch_types=[
            pltpu.VMEM((1, iters, C), jnp.int32),
            pltpu.VMEM((C, 16), F32),
            pltpu.VMEM((C, 16), F32),
            pltpu.SemaphoreType.DMA,
            pltpu.SemaphoreType.DMA,
            pltpu.SemaphoreType.DMA,
            pltpu.VMEM_SHARED((NA, 16), F32),
        ],
        compiler_params=pltpu.CompilerParams(use_tc_tiling_on_sc=False),
    )
    def scattr_k(tr_hbm, idx_hbm, z_hbm, out_hbm,
                 idx_v, t0, t1, ts0, ts1, ssem, acct):
        c = lax.axis_index("c")
        s = lax.axis_index("s")
        blk = c * 16 + s
        nslice = pl.ds(s * rows_t, rows_t)
        pltpu.sync_copy(z_hbm.at[nslice], acct.at[nslice])
        plsc.subcore_barrier()

        tb = (t0, t1)
        ts = (ts0, ts1)
        ebase = c * half + s * per_t
        pltpu.sync_copy(idx_hbm.at[pl.ds(blk, 1)], idx_v)
        pltpu.async_copy(tr_hbm.at[pl.ds(ebase, C)], tb[0], ts[0])

        def pair(g, carry):
            for b in (0, 1):
                i = 2 * g + b
                nb = 1 - b
                pltpu.make_async_copy(
                    tr_hbm.at[pl.ds(ebase + i * C, C)], tb[b], ts[b]).wait()

                @pl.when(i + 1 < iters)
                def _():
                    pltpu.async_copy(
                        tr_hbm.at[pl.ds(ebase + (i + 1) * C, C)],
                        tb[nb], ts[nb])

                pltpu.sync_copy(tb[b], acct.at[idx_v.at[0, i]], add=True)
            return carry

        lax.fori_loop(0, iters // 2, pair, 0)
        plsc.subcore_barrier()
        pltpu.sync_copy(acct.at[nslice], out_hbm.at[c].at[nslice])

    return scattr_k(tr16, row3, z16)


def _node_body(aggc_ref, h_ref, aggr_ref, aggt_ref, coord_ref, vel_ref,
               wn1ot_ref, wn1ht_ref, wn1at_ref, bn1_ref, wn2t_ref, bn2_ref,
               wv1t_ref, bv1_ref, wv2t_ref, bv2_ref,
               hout_ref, cout_ref, vout_ref):
    h = h_ref[...]
    aggt = aggt_ref[0] + aggt_ref[1]
    cnt = jnp.maximum(aggt[:, 3:4], 1.0)
    f = aggt[:, 0:3] / cnt
    hv = jnp.maximum(
        jnp.dot(h, wv1t_ref[...], preferred_element_type=F32) + bv1_ref[...],
        0.0)
    phi = jnp.dot(hv, wv2t_ref[...], preferred_element_type=F32) + bv2_ref[...]
    vel_new = phi * vel_ref[...] + f
    vout_ref[...] = vel_new
    cout_ref[...] = coord_ref[...] + vel_new
    pre = (jnp.dot(aggc_ref[...], wn1ot_ref[...], preferred_element_type=F32)
           + jnp.dot(h, wn1ht_ref[...], preferred_element_type=F32)
           + jnp.dot(aggr_ref[...], wn1at_ref[...], preferred_element_type=F32)
           + bn1_ref[...])
    hn = jnp.dot(jnp.maximum(pre, 0.0), wn2t_ref[...],
                 preferred_element_type=F32) + bn2_ref[...]
    hout_ref[...] = h + hn


def _node_update(aggc, h, aggr, aggt2, coord, vel, wn1ot, wn1ht, wn1at, bn1,
                 wn2t, bn2, wv1t, bv1, wv2t, bv2, N, BN):
    nb = N // BN
    wspec = lambda shape: pl.BlockSpec(shape, lambda i: tuple(0 for _ in shape))
    return pl.pallas_call(
        _node_body,
        grid=(nb,),
        in_specs=[
            pl.BlockSpec((BN, 128), lambda i: (i, 0)),
            pl.BlockSpec((BN, 128), lambda i: (i, 0)),
            pl.BlockSpec((BN, 128), lambda i: (i, 0)),
            pl.BlockSpec((2, BN, 16), lambda i: (0, i, 0)),
            pl.BlockSpec((BN, 3), lambda i: (i, 0)),
            pl.BlockSpec((BN, 3), lambda i: (i, 0)),
            wspec((128, 128)),
            wspec((128, 128)),
            wspec((128, 128)),
            wspec((1, 128)),
            wspec((128, 128)),
            wspec((1, 128)),
            wspec((128, 128)),
            wspec((1, 128)),
            wspec((128, 1)),
            wspec((1, 1)),
        ],
        out_specs=[
            pl.BlockSpec((BN, 128), lambda i: (i, 0)),
            pl.BlockSpec((BN, 3), lambda i: (i, 0)),
            pl.BlockSpec((BN, 3), lambda i: (i, 0)),
        ],
        out_shape=[
            jax.ShapeDtypeStruct((N, 128), F32),
            jax.ShapeDtypeStruct((N, 3), F32),
            jax.ShapeDtypeStruct((N, 3), F32),
        ],
    )(aggc, h, aggr, aggt2, coord, vel, wn1ot, wn1ht, wn1at, bn1, wn2t, bn2,
      wv1t, bv1, wv2t, bv2)


def kernel(h, edge_index, coord, vel, edge_attr, We1, be1, We2, be2,
           Wn1, bn1, Wn2, bn2, Wc1, bc1, Wc2, Wv1, bv1, Wv2, bv2):
    N, D = h.shape
    E = edge_index.shape[1]
    BN = 2000
    BE = 4096
    EP = ((E + 8191) // 8192) * 8192
    PAD = EP - E
    NA = N + 16

    row = edge_index[0].astype(jnp.int32)
    col = edge_index[1].astype(jnp.int32)
    rowp = jnp.pad(row, (0, PAD))
    colp = jnp.pad(col, (0, PAD))
    rows_s = jnp.pad(row, (0, PAD), constant_values=N)
    cols_s = jnp.pad(col, (0, PAD), constant_values=N)

    w_stk = jnp.stack([We1[:, :D].T, We1[:, D:2 * D].T])
    b_stk = jnp.stack([jnp.zeros((1, 128), F32), be1.reshape(1, 128)])
    we1r = We1[:, 2 * D:2 * D + 1].T
    we1et = We1[:, 2 * D + 1:].T
    c16 = jnp.pad(coord, ((0, 0), (0, 13)))
    ea_p = jnp.pad(edge_attr, ((0, PAD), (0, 0)))

    table2 = _precompute_tables(h, w_stk, b_stk, N, BN)

    cat_idx = jnp.concatenate([rowp, colp + N]).reshape(32, -1, C)
    cat_rc = jnp.concatenate([rowp, colp]).reshape(32, -1, C)
    gp = _ring_gather(table2, cat_idx, 2 * EP, 128, True)
    g16 = _ring_gather(c16, cat_rc, 2 * EP, 16, False)

    feat, tr16 = _edge_mlp(gp, g16, ea_p, we1r, we1et, We2.T,
                           be2.reshape(1, 128), Wc1.T, bc1.reshape(1, 128),
                           Wc2.T, EP, BE)

    z128 = jnp.zeros((NA, 128), F32)
    z16 = jnp.zeros((NA, 16), F32)
    cat_s = jnp.concatenate([rows_s, cols_s]).reshape(32, -1, C)
    aggt2 = _segment_tr(tr16, rows_s.reshape(32, -1, C), z16, NA, EP)
    aggr, aggc = _segment_feat(feat, cat_s, z128, NA, EP)

    h_new, coord_new, vel_new = _node_update(
        aggc, h, aggr, aggt2, coord, vel,
        Wn1[:, :128].T, Wn1[:, 128:256].T, Wn1[:, 256:].T,
        bn1.reshape(1, 128), Wn2.T, bn2.reshape(1, 128),
        Wv1.T, bv1.reshape(1, 128), Wv2.T, bv2.reshape(1, 1), N, BN)
    return h_new, coord_new, vel_new


# speedup vs baseline: 3.4752x; 1.0309x over previous
import functools

import jax
import jax.numpy as jnp
from jax import lax
from jax.experimental import pallas as pl
from jax.experimental.pallas import tpu as pltpu
from jax.experimental.pallas import tpu_sc as plsc

F32 = jnp.float32
C = 128


def _pre_body(h_ref, w_ref, b_ref, out_ref):
    out_ref[...] = jnp.dot(h_ref[...], w_ref[0],
                           preferred_element_type=F32) + b_ref[0]


def _precompute_tables(h, w_stk, b_stk, N, BN):
    nb = N // BN
    return pl.pallas_call(
        _pre_body,
        grid=(2, nb),
        in_specs=[
            pl.BlockSpec((BN, 128), lambda j, i: (i, 0)),
            pl.BlockSpec((1, 128, 128), lambda j, i: (j, 0, 0)),
            pl.BlockSpec((1, 1, 128), lambda j, i: (j, 0, 0)),
        ],
        out_specs=pl.BlockSpec((BN, 128), lambda j, i: (j * nb + i, 0)),
        out_shape=jax.ShapeDtypeStruct((2 * N, 128), F32),
    )(h, w_stk, b_stk)


def _ring_gather(table, idx3d, E2, W, tc_tiling):
    NW = 32
    per_w = E2 // NW
    iters = per_w // C
    mesh = plsc.VectorSubcoreMesh(core_axis_name="c", subcore_axis_name="s")

    @functools.partial(
        pl.kernel,
        mesh=mesh,
        out_type=jax.ShapeDtypeStruct((E2, W), F32),
        scratch_types=[
            pltpu.VMEM((1, iters, C), jnp.int32),
            pltpu.VMEM((C, W), F32),
            pltpu.VMEM((C, W), F32),
            pltpu.VMEM((C, W), F32),
            pltpu.VMEM((C, W), F32),
            pltpu.SemaphoreType.DMA,
            pltpu.SemaphoreType.DMA,
            pltpu.SemaphoreType.DMA,
            pltpu.SemaphoreType.DMA,
            pltpu.SemaphoreType.DMA,
            pltpu.SemaphoreType.DMA,
            pltpu.SemaphoreType.DMA,
            pltpu.SemaphoreType.DMA,
        ],
        compiler_params=pltpu.CompilerParams(use_tc_tiling_on_sc=tc_tiling),
    )
    def gather_k(table_hbm, idx_hbm, out_hbm, idx_v,
                 r0, r1, r2, r3, g0, g1, g2, g3, w0, w1, w2, w3):
        wid = lax.axis_index("c") * 16 + lax.axis_index("s")
        base = wid * per_w
        rows = (r0, r1, r2, r3)
        gs = (g0, g1, g2, g3)
        ws = (w0, w1, w2, w3)
        pltpu.sync_copy(idx_hbm.at[pl.ds(wid, 1)], idx_v)
        for b in (0, 1, 2):
            pltpu.async_copy(table_hbm.at[idx_v.at[0, b]], rows[b], gs[b])

        def quad(g, carry):
            for b in (0, 1, 2, 3):
                i = 4 * g + b
                nb = (b + 3) % 4
                pltpu.make_async_copy(
                    table_hbm.at[idx_v.at[0, i]], rows[b], gs[b]).wait()

                @pl.when((i >= 1) & (i + 3 < iters))
                def _():
                    pltpu.make_async_copy(
                        rows[nb], out_hbm.at[pl.ds(base + (i - 1) * C, C)],
                        ws[nb]).wait()

                @pl.when(i + 3 < iters)
                def _():
                    pltpu.async_copy(
                        table_hbm.at[idx_v.at[0, i + 3]], rows[nb], gs[nb])

                pltpu.async_copy(
                    rows[b], out_hbm.at[pl.ds(base + i * C, C)], ws[b])
            return carry

        lax.fori_loop(0, iters // 4, quad, 0)
        for k in range(4):
            i = iters - 4 + k
            pltpu.make_async_copy(
                rows[i % 4], out_hbm.at[pl.ds(base + i * C, C)],
                ws[i % 4]).wait()

    return gather_k(table, idx3d)


def _edge_body(gpr_ref, gpc_ref, gcr_ref, gcc_ref, ea_ref, we1r_ref,
               we1et_ref, we2t_ref, be2_ref, wc1t_ref, bc1_ref, wc2t_ref,
               feat_ref, tr_ref):
    a = gpr_ref[...] + gpc_ref[...]
    cd = gcr_ref[...] - gcc_ref[...]
    radial = jnp.sum(cd * cd, axis=1, keepdims=True)
    ef = a + radial * we1r_ref[...] + jnp.dot(
        ea_ref[...], we1et_ref[...], preferred_element_type=F32)
    ef = jnp.maximum(ef, 0.0)
    feat = jnp.maximum(
        jnp.dot(ef, we2t_ref[...], preferred_element_type=F32) + be2_ref[...],
        0.0)
    feat_ref[...] = feat
    cm = jnp.maximum(
        jnp.dot(feat, wc1t_ref[...], preferred_element_type=F32) + bc1_ref[...],
        0.0)
    m = jnp.dot(cm, wc2t_ref[...], preferred_element_type=F32)
    tr = jnp.clip(cd * m, -100.0, 100.0)
    one3 = (lax.broadcasted_iota(jnp.int32, (1, 16), 1) == 3).astype(F32)
    tr_ref[...] = tr + one3


def _edge_mlp(gp, g16, ea, we1r, we1et, we2t, be2, wc1t, bc1, wc2t,
              H, BE, s, nall):
    nb = H // BE
    wspec = lambda shape: pl.BlockSpec(shape, lambda i: tuple(0 for _ in shape))
    return pl.pallas_call(
        _edge_body,
        grid=(nb,),
        in_specs=[
            pl.BlockSpec((BE, 128), lambda i: (i, 0)),
            pl.BlockSpec((BE, 128), lambda i: (i + nb, 0)),
            pl.BlockSpec((BE, 16), lambda i: (s * nb + i, 0)),
            pl.BlockSpec((BE, 16), lambda i: (nall + s * nb + i, 0)),
            pl.BlockSpec((BE, 16), lambda i: (s * nb + i, 0)),
            wspec((1, 128)),
            wspec((16, 128)),
            wspec((128, 128)),
            wspec((1, 128)),
            wspec((128, 128)),
            wspec((1, 128)),
            wspec((128, 1)),
        ],
        out_specs=[
            pl.BlockSpec((BE, 128), lambda i: (i, 0)),
            pl.BlockSpec((BE, 16), lambda i: (i, 0)),
        ],
        out_shape=[
            jax.ShapeDtypeStruct((H, 128), F32),
            jax.ShapeDtypeStruct((H, 16), F32),
        ],
    )(gp, gp, g16, g16, ea, we1r, we1et, we2t, be2, wc1t, bc1, wc2t)


def _segment_feat(feat, cat_rc3, z128, NA, EP):
    NT = 16
    per_t = EP // NT
    iters = per_t // C
    HI = iters // 2
    mesh = plsc.VectorSubcoreMesh(core_axis_name="c", subcore_axis_name="s")

    @functools.partial(
        pl.kernel,
        mesh=mesh,
        out_type=(
            jax.ShapeDtypeStruct((NA, 128), F32),
            jax.ShapeDtypeStruct((NA, 128), F32),
        ),
        scratch_types=[
            pltpu.VMEM((1, HI, C), jnp.int32),
            pltpu.VMEM((C, 128), F32),
            pltpu.VMEM((C, 128), F32),
            pltpu.SemaphoreType.DMA,
            pltpu.SemaphoreType.DMA,
            pltpu.SemaphoreType.DMA,
            pltpu.VMEM_SHARED((NA, 128), F32),
        ],
    )
    def scatter_k(feat_hbm, idx_hbm, z_hbm, outr_hbm, outc_hbm,
                  idx_v, f0, f1, fs0, fs1, ssem, accf):
        c = lax.axis_index("c")
        s = lax.axis_index("s")
        blk = c * 16 + s
        base = s * 640
        tail = NA - 15 * 640

        @pl.when(s < 15)
        def _():
            sl = pl.ds(base, 640)
            pltpu.sync_copy(z_hbm.at[sl], accf.at[sl])

        @pl.when(s == 15)
        def _():
            sl = pl.ds(base, tail)
            pltpu.sync_copy(z_hbm.at[sl], accf.at[sl])

        plsc.subcore_barrier()

        fb = (f0, f1)
        fs = (fs0, fs1)
        ebase = s * per_t

        for h in (0, 1):
            pltpu.sync_copy(idx_hbm.at[pl.ds(blk, 1), pl.ds(h * HI, HI)],
                            idx_v)
            hbase = ebase + h * HI * C
            pltpu.async_copy(feat_hbm.at[pl.ds(hbase, C)], fb[0], fs[0])

            def pair(g, carry):
                for b in (0, 1):
                    i = 2 * g + b
                    nb = 1 - b
                    pltpu.make_async_copy(
                        feat_hbm.at[pl.ds(hbase + i * C, C)],
                        fb[b], fs[b]).wait()

                    @pl.when(i + 1 < HI)
                    def _():
                        pltpu.async_copy(
                            feat_hbm.at[pl.ds(hbase + (i + 1) * C, C)],
                            fb[nb], fs[nb])

                    pltpu.sync_copy(fb[b], accf.at[idx_v.at[0, i]], add=True)
                return carry

            lax.fori_loop(0, HI // 2, pair, 0)

        plsc.subcore_barrier()

        @pl.when((c == 0) & (s < 15))
        def _():
            sl = pl.ds(base, 640)
            pltpu.sync_copy(accf.at[sl], outr_hbm.at[sl])

        @pl.when((c == 0) & (s == 15))
        def _():
            sl = pl.ds(base, tail)
            pltpu.sync_copy(accf.at[sl], outr_hbm.at[sl])

        @pl.when((c == 1) & (s < 15))
        def _():
            sl = pl.ds(base, 640)
            pltpu.sync_copy(accf.at[sl], outc_hbm.at[sl])

        @pl.when((c == 1) & (s == 15))
        def _():
            sl = pl.ds(base, tail)
            pltpu.sync_copy(accf.at[sl], outc_hbm.at[sl])

    return scatter_k(feat, cat_rc3, z128)


def _segment_tr(tr16, row3, z16, NA, EP):
    NT = 16
    half = EP // 2
    per_t = half // NT
    iters = per_t // C
    rows_t = NA // NT
    mesh = plsc.VectorSubcoreMesh(core_axis_name="c", subcore_axis_name="s")

    @functools.partial(
        pl.kernel,
        mesh=mesh,
        out_type=jax.ShapeDtypeStruct((2, NA, 16), F32),
        scratch_types=[
            pltpu.VMEM((1, iters, C), jnp.int32),
            pltpu.VMEM((C, 16), F32),
            pltpu.VMEM((C, 16), F32),
            pltpu.SemaphoreType.DMA,
            pltpu.SemaphoreType.DMA,
            pltpu.SemaphoreType.DMA,
            pltpu.VMEM_SHARED((NA, 16), F32),
        ],
        compiler_params=pltpu.CompilerParams(use_tc_tiling_on_sc=False),
    )
    def scattr_k(tr_hbm, idx_hbm, z_hbm, out_hbm,
                 idx_v, t0, t1, ts0, ts1, ssem, acct):
        c = lax.axis_index("c")
        s = lax.axis_index("s")
        blk = c * 16 + s
        nslice = pl.ds(s * rows_t, rows_t)
        pltpu.sync_copy(z_hbm.at[nslice], acct.at[nslice])
        plsc.subcore_barrier()

        tb = (t0, t1)
        ts = (ts0, ts1)
        ebase = c * half + s * per_t
        pltpu.sync_copy(idx_hbm.at[pl.ds(blk, 1)], idx_v)
        pltpu.async_copy(tr_hbm.at[pl.ds(ebase, C)], tb[0], ts[0])

        def pair(g, carry):
            for b in (0, 1):
                i = 2 * g + b
                nb = 1 - b
                pltpu.make_async_copy(
                    tr_hbm.at[pl.ds(ebase + i * C, C)], tb[b], ts[b]).wait()

                @pl.when(i + 1 < iters)
                def _():
                    pltpu.async_copy(
                        tr_hbm.at[pl.ds(ebase + (i + 1) * C, C)],
                        tb[nb], ts[nb])

                pltpu.sync_copy(tb[b], acct.at[idx_v.at[0, i]], add=True)
            return carry

        lax.fori_loop(0, iters // 2, pair, 0)
        plsc.subcore_barrier()
        pltpu.sync_copy(acct.at[nslice], out_hbm.at[c].at[nslice])

    return scattr_k(tr16, row3, z16)


def _node_body(aggc0_ref, aggc1_ref, h_ref, aggr0_ref, aggr1_ref,
               aggta_ref, aggtb_ref, coord_ref, vel_ref,
               wn1ot_ref, wn1ht_ref, wn1at_ref, bn1_ref, wn2t_ref, bn2_ref,
               wv1t_ref, bv1_ref, wv2t_ref, bv2_ref,
               hout_ref, cout_ref, vout_ref):
    h = h_ref[...]
    aggt = (aggta_ref[0] + aggta_ref[1]) + (aggtb_ref[0] + aggtb_ref[1])
    cnt = jnp.maximum(aggt[:, 3:4], 1.0)
    f = aggt[:, 0:3] / cnt
    hv = jnp.maximum(
        jnp.dot(h, wv1t_ref[...], preferred_element_type=F32) + bv1_ref[...],
        0.0)
    phi = jnp.dot(hv, wv2t_ref[...], preferred_element_type=F32) + bv2_ref[...]
    vel_new = phi * vel_ref[...] + f
    vout_ref[...] = vel_new
    cout_ref[...] = coord_ref[...] + vel_new
    aggc = aggc0_ref[...] + aggc1_ref[...]
    aggr = aggr0_ref[...] + aggr1_ref[...]
    pre = (jnp.dot(aggc, wn1ot_ref[...], preferred_element_type=F32)
           + jnp.dot(h, wn1ht_ref[...], preferred_element_type=F32)
           + jnp.dot(aggr, wn1at_ref[...], preferred_element_type=F32)
           + bn1_ref[...])
    hn = jnp.dot(jnp.maximum(pre, 0.0), wn2t_ref[...],
                 preferred_element_type=F32) + bn2_ref[...]
    hout_ref[...] = h + hn


def _node_update(aggc0, aggc1, h, aggr0, aggr1, aggta, aggtb, coord, vel,
                 wn1ot, wn1ht, wn1at, bn1,
                 wn2t, bn2, wv1t, bv1, wv2t, bv2, N, BN):
    nb = N // BN
    wspec = lambda shape: pl.BlockSpec(shape, lambda i: tuple(0 for _ in shape))
    return pl.pallas_call(
        _node_body,
        grid=(nb,),
        in_specs=[
            pl.BlockSpec((BN, 128), lambda i: (i, 0)),
            pl.BlockSpec((BN, 128), lambda i: (i, 0)),
            pl.BlockSpec((BN, 128), lambda i: (i, 0)),
            pl.BlockSpec((BN, 128), lambda i: (i, 0)),
            pl.BlockSpec((BN, 128), lambda i: (i, 0)),
            pl.BlockSpec((2, BN, 16), lambda i: (0, i, 0)),
            pl.BlockSpec((2, BN, 16), lambda i: (0, i, 0)),
            pl.BlockSpec((BN, 3), lambda i: (i, 0)),
            pl.BlockSpec((BN, 3), lambda i: (i, 0)),
            wspec((128, 128)),
            wspec((128, 128)),
            wspec((128, 128)),
            wspec((1, 128)),
            wspec((128, 128)),
            wspec((1, 128)),
            wspec((128, 128)),
            wspec((1, 128)),
            wspec((128, 1)),
            wspec((1, 1)),
        ],
        out_specs=[
            pl.BlockSpec((BN, 128), lambda i: (i, 0)),
            pl.BlockSpec((BN, 3), lambda i: (i, 0)),
            pl.BlockSpec((BN, 3), lambda i: (i, 0)),
        ],
        out_shape=[
            jax.ShapeDtypeStruct((N, 128), F32),
            jax.ShapeDtypeStruct((N, 3), F32),
            jax.ShapeDtypeStruct((N, 3), F32),
        ],
    )(aggc0, aggc1, h, aggr0, aggr1, aggta, aggtb, coord, vel,
      wn1ot, wn1ht, wn1at, bn1, wn2t, bn2, wv1t, bv1, wv2t, bv2)


def kernel(h, edge_index, coord, vel, edge_attr, We1, be1, We2, be2,
           Wn1, bn1, Wn2, bn2, Wc1, bc1, Wc2, Wv1, bv1, Wv2, bv2):
    N, D = h.shape
    E = edge_index.shape[1]
    BN = 2000
    BE = 4096
    EP = ((E + 8191) // 8192) * 8192
    PAD = EP - E
    NA = N + 16

    row = edge_index[0].astype(jnp.int32)
    col = edge_index[1].astype(jnp.int32)
    rowp = jnp.pad(row, (0, PAD))
    colp = jnp.pad(col, (0, PAD))
    rows_s = jnp.pad(row, (0, PAD), constant_values=N)
    cols_s = jnp.pad(col, (0, PAD), constant_values=N)

    w_stk = jnp.stack([We1[:, :D].T, We1[:, D:2 * D].T])
    b_stk = jnp.stack([jnp.zeros((1, 128), F32), be1.reshape(1, 128)])
    we1r = We1[:, 2 * D:2 * D + 1].T
    we1et = We1[:, 2 * D + 1:].T
    c16 = jnp.pad(coord, ((0, 0), (0, 13)))
    ea_p = jnp.pad(edge_attr, ((0, PAD), (0, 0)))

    table2 = _precompute_tables(h, w_stk, b_stk, N, BN)

    H = EP // 2
    cat_rc = jnp.concatenate([rowp, colp]).reshape(32, -1, C)
    g16 = _ring_gather(c16, cat_rc, 2 * EP, 16, False)
    gps = []
    for s in (0, 1):
        ci = jnp.concatenate(
            [rowp[s * H:(s + 1) * H],
             colp[s * H:(s + 1) * H] + N]).reshape(32, -1, C)
        gps.append(_ring_gather(table2, ci, 2 * H, 128, True))

    z128 = jnp.zeros((NA, 128), F32)
    z16 = jnp.zeros((NA, 16), F32)
    aggrs, aggcs, aggts = [], [], []
    for s in (0, 1):
        feat_s, tr_s = _edge_mlp(gps[s], g16, ea_p, we1r, we1et, We2.T,
                                 be2.reshape(1, 128), Wc1.T,
                                 bc1.reshape(1, 128), Wc2.T,
                                 H, BE, s, EP // BE)
        cs = jnp.concatenate(
            [rows_s[s * H:(s + 1) * H],
             cols_s[s * H:(s + 1) * H]]).reshape(32, -1, C)
        aggts.append(_segment_tr(
            tr_s, rows_s[s * H:(s + 1) * H].reshape(32, -1, C), z16, NA, H))
        r_, c_ = _segment_feat(feat_s, cs, z128, NA, H)
        aggrs.append(r_)
        aggcs.append(c_)

    h_new, coord_new, vel_new = _node_update(
        aggcs[0], aggcs[1], h, aggrs[0], aggrs[1], aggts[0], aggts[1],
        coord, vel,
        Wn1[:, :128].T, Wn1[:, 128:256].T, Wn1[:, 256:].T,
        bn1.reshape(1, 128), Wn2.T, bn2.reshape(1, 128),
        Wv1.T, bv1.reshape(1, 128), Wv2.T, bv2.reshape(1, 1), N, BN)
    return h_new, coord_new, vel_new
